# Initial kernel scaffold; baseline (speedup 1.0000x reference)
import jax
import jax.numpy as jnp
from jax.experimental import pallas as pl


def kernel(uv_src, uv_dst, uv_rating, uu_src, uu_dst, params):
    raise NotImplementedError("write your pallas kernel here")


# speedup vs baseline: 2.4081x; 2.4081x over previous
import functools

import jax
import jax.numpy as jnp
from jax import lax
from jax.experimental import pallas as pl
from jax.experimental.pallas import tpu as pltpu
from jax.experimental.pallas import tpu_sc as plsc

N_NODE = 10000
D = 128
NR = 5
NV = NR * N_NODE
NC, NS = 2, 16
NW = NC * NS
NPAD = 10240
ZB = 16


def _lrelu(x):
    return jnp.where(x > 0, x, 0.2 * x)


def _bcast(vec16, j):
    idx = jnp.full((16, 1), j, jnp.int32)
    return lax.gather(
        vec16, idx,
        lax.GatherDimensionNumbers(
            offset_dims=(), collapsed_slice_dims=(0,), start_index_map=(0,)),
        (1,), mode=lax.GatherScatterMode.PROMISE_IN_BOUNDS)


def _pre_side(node_emb, r_emb, w1, b1, w2, b2, a1h, act):
    BM = 400
    nblk = N_NODE // BM

    def body(nb_ref, re_ref, w1_ref, b1_ref, w2_ref, b2_ref, a1_ref,
             x_ref, a_ref):
        w1a = w1_ref[:D, :]
        w1b = w1_ref[D:, :]
        rp = re_ref[...] @ w1b
        t = nb_ref[...] @ w1a
        b1v = b1_ref[...]
        for r in range(NR):
            x1 = act(t + rp[r][None, :] + b1v)
            xa = act(x1 @ w2_ref[...] + b2_ref[...])
            x_ref[r] = xa
            a_ref[r] = xa @ a1_ref[...]

    x_all, a_all = pl.pallas_call(
        body,
        grid=(nblk,),
        in_specs=[
            pl.BlockSpec((BM, D), lambda i: (i, 0)),
            pl.BlockSpec((NR, D), lambda i: (0, 0)),
            pl.BlockSpec((2 * D, D), lambda i: (0, 0)),
            pl.BlockSpec((1, D), lambda i: (0, 0)),
            pl.BlockSpec((D, D), lambda i: (0, 0)),
            pl.BlockSpec((1, D), lambda i: (0, 0)),
            pl.BlockSpec((D, D), lambda i: (0, 0)),
        ],
        out_specs=[
            pl.BlockSpec((NR, BM, D), lambda i: (0, i, 0)),
            pl.BlockSpec((NR, BM, D), lambda i: (0, i, 0)),
        ],
        out_shape=[
            jax.ShapeDtypeStruct((NR, N_NODE, D), jnp.float32),
            jax.ShapeDtypeStruct((NR, N_NODE, D), jnp.float32),
        ],
    )(node_emb, r_emb, w1, b1.reshape(1, D), w2, b2.reshape(1, D), a1h)
    return x_all.reshape(NV, D), a_all.reshape(NV, D)


def _mm(xs, ws, b, act):
    n = xs[0].shape[0]
    BM = 400 if n % 400 == 0 else 320
    nblk = n // BM
    nx = len(xs)

    def body(*refs):
        x_refs = refs[:nx]
        w_refs = refs[nx:2 * nx]
        b_ref = refs[2 * nx]
        o_ref = refs[2 * nx + 1]
        acc = b_ref[...]
        for xr, wr in zip(x_refs, w_refs):
            acc = acc + xr[...] @ wr[...]
        o_ref[...] = act(acc) if act is not None else acc

    in_specs = (
        [pl.BlockSpec((BM, x.shape[1]), lambda i: (i, 0)) for x in xs]
        + [pl.BlockSpec(w.shape, lambda i: (0, 0)) for w in ws]
        + [pl.BlockSpec((1, D), lambda i: (0, 0))]
    )
    return pl.pallas_call(
        body,
        grid=(nblk,),
        in_specs=in_specs,
        out_specs=pl.BlockSpec((BM, D), lambda i: (i, 0)),
        out_shape=jax.ShapeDtypeStruct((n, D), jnp.float32),
    )(*xs, *ws, b.reshape(1, D))


def _att2(t1, a2w, a2b, a3w, a3b):
    E = t1.shape[0]
    BM = 640
    nblk = E // BM

    def body(t1_ref, w_ref, b_ref, a3_ref, a3b_ref, w_out, m_out):
        i = pl.program_id(0)
        t2 = jnp.maximum(t1_ref[...] @ w_ref[...] + b_ref[...], 0.0)
        wv = jnp.sum(t2 * a3_ref[...], axis=1) + a3b_ref[0, 0]
        w_out[...] = wv.reshape(1, 8, BM // 8)

        @pl.when(i == 0)
        def _():
            m_out[...] = jnp.full((1, D), -1e30, jnp.float32)

        m_out[...] = jnp.maximum(m_out[...], jnp.max(wv))

    wgt, m = pl.pallas_call(
        body,
        grid=(nblk,),
        in_specs=[
            pl.BlockSpec((BM, D), lambda i: (i, 0)),
            pl.BlockSpec((D, D), lambda i: (0, 0)),
            pl.BlockSpec((1, D), lambda i: (0, 0)),
            pl.BlockSpec((1, D), lambda i: (0, 0)),
            pl.BlockSpec((1, 1), lambda i: (0, 0)),
        ],
        out_specs=[
            pl.BlockSpec((1, 8, BM // 8), lambda i: (i, 0, 0)),
            pl.BlockSpec((1, D), lambda i: (0, 0)),
        ],
        out_shape=[
            jax.ShapeDtypeStruct((nblk, 8, BM // 8), jnp.float32),
            jax.ShapeDtypeStruct((1, D), jnp.float32),
        ],
    )(t1, a2w, a2b.reshape(1, D), a3w.reshape(1, D), a3b.reshape(1, 1))
    return wgt.reshape(E), m.reshape(D)


@functools.partial(jax.jit, static_argnames=("C",))
def _sc_gather_relu(A, B, ia, ib, C):
    E = ia.shape[0]
    PW = E // NW
    nch = PW // C
    mesh = plsc.VectorSubcoreMesh(core_axis_name="c", subcore_axis_name="s", num_cores=NC, num_subcores=NS)

    @functools.partial(
        pl.kernel,
        out_type=jax.ShapeDtypeStruct((E, D), jnp.float32),
        mesh=mesh,
        scratch_types=[
            pltpu.VMEM((C,), jnp.int32),
            pltpu.VMEM((C,), jnp.int32),
            pltpu.VMEM((C, D), jnp.float32),
            pltpu.VMEM((C, D), jnp.float32),
            pltpu.SemaphoreType.DMA,
            pltpu.SemaphoreType.DMA,
        ],
    )
    def k(A_h, B_h, ia_h, ib_h, out_h, iav, ibv, rowsA, rowsB, semA, semB):
        c = lax.axis_index("c")
        s = lax.axis_index("s")
        base = (s * NC + c) * PW

        def chunk(i, _):
            off = base + i * C
            pltpu.sync_copy(ia_h.at[pl.ds(off, C)], iav)
            pltpu.sync_copy(ib_h.at[pl.ds(off, C)], ibv)
            cpA = pltpu.async_copy(A_h.at[iav], rowsA, semA)
            cpB = pltpu.async_copy(B_h.at[ibv], rowsB, semB)
            cpA.wait()
            cpB.wait()

            def rowf(r, _):
                for j in range(D // 16):
                    sl = (r, pl.ds(j * 16, 16))
                    rowsA[sl] = jnp.maximum(rowsA[sl] + rowsB[sl], 0.0)
                return 0

            lax.fori_loop(0, C, rowf, 0)
            pltpu.sync_copy(rowsA, out_h.at[pl.ds(off, C)])
            return 0

        lax.fori_loop(0, nch, chunk, 0)

    return k(A, B, ia, ib)


@functools.partial(jax.jit, static_argnames=("C",))
def _sc_seg_sum(wgt, mvec, seg, C):
    E = wgt.shape[0]
    PW = E // NW
    nch = PW // C
    mesh = plsc.VectorSubcoreMesh(core_axis_name="c", subcore_axis_name="s", num_cores=NC, num_subcores=NS)

    @functools.partial(
        pl.kernel,
        out_type=[
            jax.ShapeDtypeStruct((NPAD, D), jnp.float32),
            jax.ShapeDtypeStruct((NPAD, D), jnp.float32),
        ],
        mesh=mesh,
        scratch_types=[
            pltpu.VMEM_SHARED((NPAD, D), jnp.float32),
            pltpu.VMEM((16,), jnp.float32),
            pltpu.VMEM((C,), jnp.float32),
            pltpu.VMEM((C, D), jnp.float32),
            pltpu.VMEM((C,), jnp.int32),
            pltpu.VMEM((1, 16), jnp.float32),
        ],
    )
    def k(wgt_h, m_h, seg_h, out0_h, out1_h, s_sh, mv, wv, wrows, segv, onev):
        c = lax.axis_index("c")
        s = lax.axis_index("s")
        zero16 = jnp.zeros((16,), jnp.float32)

        def z1(r, _):
            for j in range(D // 16):
                wrows[r, pl.ds(j * 16, 16)] = zero16
            return 0

        lax.fori_loop(0, C, z1, 0)
        onev[0, pl.ds(0, 16)] = jnp.ones((16,), jnp.float32)

        def z3(kk, _):
            r0 = s * (NPAD // NS) + kk * ZB
            pltpu.sync_copy(wrows.at[pl.ds(0, ZB)], s_sh.at[pl.ds(r0, ZB)])
            return 0

        lax.fori_loop(0, NPAD // NS // ZB, z3, 0)
        pltpu.sync_copy(m_h.at[pl.ds(0, 16)], mv)
        plsc.subcore_barrier()

        mvv = mv[...]
        base = (s * NC + c) * PW

        def pha(i, _):
            off = base + i * C
            pltpu.sync_copy(wgt_h.at[pl.ds(off, C)], wv)
            pltpu.sync_copy(seg_h.at[pl.ds(off, C)], segv)

            def grp(g, _):
                sl = pl.ds(g * 16, 16)
                wv[sl] = jnp.exp(wv[sl] - mvv)
                return 0

            lax.fori_loop(0, C // 16, grp, 0)

            def rowb(r, _):
                grpv = wv[pl.ds((r // 16) * 16, 16)]
                bc = onev[0, pl.ds(0, 16)] * _bcast(grpv, r % 16)
                for j in range(D // 16):
                    wrows[r, pl.ds(j * 16, 16)] = bc
                return 0

            lax.fori_loop(0, C, rowb, 0)
            pltpu.sync_copy(wrows, s_sh.at[segv], add=True)
            return 0

        lax.fori_loop(0, nch, pha, 0)
        plsc.subcore_barrier()

        def dump(kk, _):
            r0 = s * (NPAD // NS) + kk * ZB
            pltpu.sync_copy(s_sh.at[pl.ds(r0, ZB)], wrows.at[pl.ds(0, ZB)])

            @pl.when(c == 0)
            def _():
                pltpu.sync_copy(wrows.at[pl.ds(0, ZB)], out0_h.at[pl.ds(r0, ZB)])

            @pl.when(c == 1)
            def _():
                pltpu.sync_copy(wrows.at[pl.ds(0, ZB)], out1_h.at[pl.ds(r0, ZB)])

            return 0

        lax.fori_loop(0, NPAD // NS // ZB, dump, 0)

    return k(wgt, mvec, seg)


@functools.partial(jax.jit, static_argnames=("C",))
def _sc_agg(wgt, mvec, X, seg, key, s0, s1, C):
    E = wgt.shape[0]
    PW = E // NW
    nch = PW // C
    mesh = plsc.VectorSubcoreMesh(core_axis_name="c", subcore_axis_name="s", num_cores=NC, num_subcores=NS)

    @functools.partial(
        pl.kernel,
        out_type=[
            jax.ShapeDtypeStruct((NPAD, D), jnp.float32),
            jax.ShapeDtypeStruct((NPAD, D), jnp.float32),
        ],
        mesh=mesh,
        scratch_types=[
            pltpu.VMEM_SHARED((NPAD, D), jnp.float32),
            pltpu.VMEM((16,), jnp.float32),
            pltpu.VMEM((C,), jnp.float32),
            pltpu.VMEM((C, D), jnp.float32),
            pltpu.VMEM((ZB, D), jnp.float32),
            pltpu.VMEM((ZB, D), jnp.float32),
            pltpu.VMEM((C,), jnp.int32),
            pltpu.VMEM((C,), jnp.int32),
            pltpu.SemaphoreType.DMA,
        ],
    )
    def k(wgt_h, m_h, X_h, seg_h, key_h, s0_h, s1_h, out0_h, out1_h,
          h_sh, mv, wv, xrows, sa, sb, segv, keyv, sem):
        c = lax.axis_index("c")
        s = lax.axis_index("s")
        zero16 = jnp.zeros((16,), jnp.float32)

        def z1(r, _):
            for j in range(D // 16):
                xrows[r, pl.ds(j * 16, 16)] = zero16
            return 0

        lax.fori_loop(0, C, z1, 0)

        def z3(kk, _):
            r0 = s * (NPAD // NS) + kk * ZB
            pltpu.sync_copy(xrows.at[pl.ds(0, ZB)], h_sh.at[pl.ds(r0, ZB)])
            return 0

        lax.fori_loop(0, NPAD // NS // ZB, z3, 0)
        pltpu.sync_copy(m_h.at[pl.ds(0, 16)], mv)
        plsc.subcore_barrier()

        mvv = mv[...]
        base = (s * NC + c) * PW

        def phc(i, _):
            off = base + i * C
            pltpu.sync_copy(seg_h.at[pl.ds(off, C)], segv)
            pltpu.sync_copy(key_h.at[pl.ds(off, C)], keyv)
            pltpu.sync_copy(wgt_h.at[pl.ds(off, C)], wv)

            def mkex(g, _):
                sl = pl.ds(g * 16, 16)
                wv[sl] = jnp.exp(wv[sl] - mvv)
                return 0

            lax.fori_loop(0, C // 16, mkex, 0)
            pltpu.async_copy(X_h.at[keyv], xrows, sem).wait()

            def rowf(r, _):
                grpv = wv[pl.ds((r // 16) * 16, 16)]
                exb = _bcast(grpv, r % 16)
                for j in range(D // 16):
                    sl = (r, pl.ds(j * 16, 16))
                    xrows[sl] = xrows[sl] * exb
                return 0

            lax.fori_loop(0, C, rowf, 0)
            pltpu.sync_copy(xrows, h_sh.at[segv], add=True)
            return 0

        lax.fori_loop(0, nch, phc, 0)
        plsc.subcore_barrier()

        def dump(kk, _):
            r0 = s * (NPAD // NS) + kk * ZB
            pltpu.sync_copy(h_sh.at[pl.ds(r0, ZB)], xrows.at[pl.ds(0, ZB)])
            pltpu.sync_copy(s0_h.at[pl.ds(r0, ZB)], sa)
            pltpu.sync_copy(s1_h.at[pl.ds(r0, ZB)], sb)

            def nrm(r, _):
                for j in range(D // 16):
                    sl = (r, pl.ds(j * 16, 16))
                    xrows[sl] = xrows[sl] / (sa[sl] + sb[sl] + 1e-9)
                return 0

            lax.fori_loop(0, ZB, nrm, 0)

            @pl.when(c == 0)
            def _():
                pltpu.sync_copy(xrows.at[pl.ds(0, ZB)], out0_h.at[pl.ds(r0, ZB)])

            @pl.when(c == 1)
            def _():
                pltpu.sync_copy(xrows.at[pl.ds(0, ZB)], out1_h.at[pl.ds(r0, ZB)])

            return 0

        lax.fori_loop(0, NPAD // NS // ZB, dump, 0)

    return k(wgt, mvec, X, seg, key, s0, s1)


def _sc_softmax_agg(wgt, mvec, X, seg, key, C):
    s0, s1 = _sc_seg_sum(wgt, mvec, seg, C=C)
    return _sc_agg(wgt, mvec, X, seg, key, s0, s1, C=C)


def _jnp_softmax_agg(wgt, mvec, X, seg, key, C):
    ex = jnp.exp(wgt - mvec[0])
    s = jax.ops.segment_sum(ex, seg, num_segments=N_NODE)
    val = ex / (s[seg] + 1e-9)
    h = jax.ops.segment_sum(X[key] * val[:, None], seg, num_segments=N_NODE)
    pad = jnp.zeros((NPAD - N_NODE, D), jnp.float32)
    h = jnp.concatenate([h, pad], axis=0)
    return h, jnp.zeros_like(h)


def kernel(uv_src, uv_dst, uv_rating, uu_src, uu_dst, params):
    p = params
    relu = lambda x: jnp.maximum(x, 0.0)
    uv_src = uv_src.astype(jnp.int32)
    uv_dst = uv_dst.astype(jnp.int32)
    uv_rating = uv_rating.astype(jnp.int32)
    uu_src = uu_src.astype(jnp.int32)
    uu_dst = uu_dst.astype(jnp.int32)
    keyI = uv_rating * N_NODE + uv_dst
    keyU = uv_rating * N_NODE + uv_src

    x_ia_all, A_I = _pre_side(
        p['item_emb'], p['rating_emb'], p['gv_w1'], p['gv_b1'],
        p['gv_w2'], p['gv_b2'], p['attI_a1w'][:D], _lrelu)
    f_jt_all, A_U = _pre_side(
        p['user_emb'], p['rating_emb'], p['gu_w1'], p['gu_b1'],
        p['gu_w2'], p['gu_b2'], p['attU_a1w'][:D], relu)
    B_I = _mm([p['user_emb']], [p['attI_a1w'][D:]], p['attI_a1b'], None)
    B_U = _mm([p['item_emb']], [p['attU_a1w'][D:]], p['attU_a1b'], None)
    A_S = _mm([p['user_emb']], [p['attS_a1w'][:D]],
              jnp.zeros((D,), jnp.float32), None)

    t1 = _sc_gather_relu(A_I, B_I, keyI, uv_src, C=80)
    wgt, m = _att2(t1, p['attI_a2w'], p['attI_a2b'],
                   p['attI_a3w'], p['attI_a3b'])
    h0, h1 = _sc_softmax_agg(wgt, m, x_ia_all, uv_src, keyI, C=16)
    hI = _mm([h0, h1], [p['wi_w'], p['wi_w']], p['wi_b'], _lrelu)

    t1 = _sc_gather_relu(A_U, B_U, keyU, uv_dst, C=80)
    wgt, m = _att2(t1, p['attU_a2w'], p['attU_a2b'],
                   p['attU_a3w'], p['attU_a3b'])
    z0, z1 = _sc_softmax_agg(wgt, m, f_jt_all, uv_dst, keyU, C=16)
    z = _mm([z0, z1], [p['wu_w'], p['wu_w']], p['wu_b'], _lrelu)

    hI10k = hI[:N_NODE]
    B_S = _mm([hI10k], [p['attS_a1w'][D:]], p['attS_a1b'], None)
    t1 = _sc_gather_relu(A_S, B_S, uu_src, uu_dst, C=40)
    wgt, m = _att2(t1, p['attS_a2w'], p['attS_a2b'],
                   p['attS_a3w'], p['attS_a3b'])
    hs0, hs1 = _sc_softmax_agg(wgt, m, hI10k, uu_dst, uu_src, C=16)
    hS = _mm([hs0, hs1], [p['ws_w'], p['ws_w']], p['ws_b'], _lrelu)

    h_out = _mm([hI10k, hS[:N_NODE]], [p['w2_w'][:D], p['w2_w'][D:]],
                p['w2_b'], _lrelu)
    return (h_out, z[:N_NODE])


# speedup vs baseline: 3.3710x; 1.3999x over previous
import functools

import jax
import jax.numpy as jnp
from jax import lax
from jax.experimental import pallas as pl
from jax.experimental.pallas import tpu as pltpu
from jax.experimental.pallas import tpu_sc as plsc

N_NODE = 10000
D = 128
NR = 5
NV = NR * N_NODE
NC, NS = 2, 16
NW = NC * NS
NPAD = 10240
ZB = 16


def _lrelu(x):
    return jnp.where(x > 0, x, 0.2 * x)


def _bcast(vec16, j):
    idx = jnp.full((16, 1), j, jnp.int32)
    return lax.gather(
        vec16, idx,
        lax.GatherDimensionNumbers(
            offset_dims=(), collapsed_slice_dims=(0,), start_index_map=(0,)),
        (1,), mode=lax.GatherScatterMode.PROMISE_IN_BOUNDS)


def _pre_side(node_emb, r_emb, w1, b1, w2, b2, a1h, act):
    BM = 400
    nblk = N_NODE // BM

    def body(nb_ref, re_ref, w1_ref, b1_ref, w2_ref, b2_ref, a1_ref,
             x_ref, a_ref):
        w1a = w1_ref[:D, :]
        w1b = w1_ref[D:, :]
        rp = re_ref[...] @ w1b
        t = nb_ref[...] @ w1a
        b1v = b1_ref[...]
        for r in range(NR):
            x1 = act(t + rp[r][None, :] + b1v)
            xa = act(x1 @ w2_ref[...] + b2_ref[...])
            x_ref[r] = xa
            a_ref[r] = xa @ a1_ref[...]

    x_all, a_all = pl.pallas_call(
        body,
        grid=(nblk,),
        in_specs=[
            pl.BlockSpec((BM, D), lambda i: (i, 0)),
            pl.BlockSpec((NR, D), lambda i: (0, 0)),
            pl.BlockSpec((2 * D, D), lambda i: (0, 0)),
            pl.BlockSpec((1, D), lambda i: (0, 0)),
            pl.BlockSpec((D, D), lambda i: (0, 0)),
            pl.BlockSpec((1, D), lambda i: (0, 0)),
            pl.BlockSpec((D, D), lambda i: (0, 0)),
        ],
        out_specs=[
            pl.BlockSpec((NR, BM, D), lambda i: (0, i, 0)),
            pl.BlockSpec((NR, BM, D), lambda i: (0, i, 0)),
        ],
        out_shape=[
            jax.ShapeDtypeStruct((NR, N_NODE, D), jnp.float32),
            jax.ShapeDtypeStruct((NR, N_NODE, D), jnp.float32),
        ],
    )(node_emb, r_emb, w1, b1.reshape(1, D), w2, b2.reshape(1, D), a1h)
    return x_all.reshape(NV, D), a_all.reshape(NV, D)


def _mm(xs, ws, b, act):
    n = xs[0].shape[0]
    BM = 400 if n % 400 == 0 else 320
    nblk = n // BM
    nx = len(xs)

    def body(*refs):
        x_refs = refs[:nx]
        w_refs = refs[nx:2 * nx]
        b_ref = refs[2 * nx]
        o_ref = refs[2 * nx + 1]
        acc = b_ref[...]
        for xr, wr in zip(x_refs, w_refs):
            acc = acc + xr[...] @ wr[...]
        o_ref[...] = act(acc) if act is not None else acc

    in_specs = (
        [pl.BlockSpec((BM, x.shape[1]), lambda i: (i, 0)) for x in xs]
        + [pl.BlockSpec(w.shape, lambda i: (0, 0)) for w in ws]
        + [pl.BlockSpec((1, D), lambda i: (0, 0))]
    )
    return pl.pallas_call(
        body,
        grid=(nblk,),
        in_specs=in_specs,
        out_specs=pl.BlockSpec((BM, D), lambda i: (i, 0)),
        out_shape=jax.ShapeDtypeStruct((n, D), jnp.float32),
    )(*xs, *ws, b.reshape(1, D))


def _att2(t1, a2w, a2b, a3w, a3b):
    E = t1.shape[0]
    BM = 640
    nblk = E // BM

    def body(t1_ref, w_ref, b_ref, a3_ref, a3b_ref, w_out, m_out):
        i = pl.program_id(0)
        t2 = jnp.maximum(t1_ref[...] @ w_ref[...] + b_ref[...], 0.0)
        wv = jnp.sum(t2 * a3_ref[...], axis=1) + a3b_ref[0, 0]
        w_out[...] = wv.reshape(1, 8, BM // 8)

        @pl.when(i == 0)
        def _():
            m_out[...] = jnp.full((1, D), -1e30, jnp.float32)

        m_out[...] = jnp.maximum(m_out[...], jnp.max(wv))

    wgt, m = pl.pallas_call(
        body,
        grid=(nblk,),
        in_specs=[
            pl.BlockSpec((BM, D), lambda i: (i, 0)),
            pl.BlockSpec((D, D), lambda i: (0, 0)),
            pl.BlockSpec((1, D), lambda i: (0, 0)),
            pl.BlockSpec((1, D), lambda i: (0, 0)),
            pl.BlockSpec((1, 1), lambda i: (0, 0)),
        ],
        out_specs=[
            pl.BlockSpec((1, 8, BM // 8), lambda i: (i, 0, 0)),
            pl.BlockSpec((1, D), lambda i: (0, 0)),
        ],
        out_shape=[
            jax.ShapeDtypeStruct((nblk, 8, BM // 8), jnp.float32),
            jax.ShapeDtypeStruct((1, D), jnp.float32),
        ],
    )(t1, a2w, a2b.reshape(1, D), a3w.reshape(1, D), a3b.reshape(1, 1))
    return wgt.reshape(E), m.reshape(D)


@functools.partial(jax.jit, static_argnames=("C",))
def _sc_gather_relu(A, B, ia, ib, C):
    E = ia.shape[0]
    PW = E // NW
    nch = PW // C
    mesh = plsc.VectorSubcoreMesh(core_axis_name="c", subcore_axis_name="s", num_cores=NC, num_subcores=NS)

    @functools.partial(
        pl.kernel,
        out_type=jax.ShapeDtypeStruct((E, D), jnp.float32),
        mesh=mesh,
        scratch_types=[
            pltpu.VMEM((PW,), jnp.int32),
            pltpu.VMEM((PW,), jnp.int32),
            pltpu.VMEM((C, D), jnp.float32),
            pltpu.VMEM((C, D), jnp.float32),
            pltpu.SemaphoreType.DMA,
            pltpu.SemaphoreType.DMA,
        ],
    )
    def k(A_h, B_h, ia_h, ib_h, out_h, iav, ibv, rowsA, rowsB, semA, semB):
        c = lax.axis_index("c")
        s = lax.axis_index("s")
        base = (s * NC + c) * PW
        pltpu.sync_copy(ia_h.at[pl.ds(base, PW)], iav)
        pltpu.sync_copy(ib_h.at[pl.ds(base, PW)], ibv)

        def chunk(i, _):
            off = base + i * C
            cpA = pltpu.async_copy(A_h.at[iav.at[pl.ds(i * C, C)]], rowsA, semA)
            cpB = pltpu.async_copy(B_h.at[ibv.at[pl.ds(i * C, C)]], rowsB, semB)
            cpA.wait()
            cpB.wait()

            def rowf(r, _):
                for j in range(D // 16):
                    sl = (r, pl.ds(j * 16, 16))
                    rowsA[sl] = jnp.maximum(rowsA[sl] + rowsB[sl], 0.0)
                return 0

            lax.fori_loop(0, C, rowf, 0)
            pltpu.sync_copy(rowsA, out_h.at[pl.ds(off, C)])
            return 0

        lax.fori_loop(0, nch, chunk, 0)

    return k(A, B, ia, ib)


@functools.partial(jax.jit, static_argnames=("C",))
def _sc_seg_sum(wgt, mvec, seg, C):
    E = wgt.shape[0]
    PW = E // NW
    nch = PW // C
    mesh = plsc.VectorSubcoreMesh(core_axis_name="c", subcore_axis_name="s", num_cores=NC, num_subcores=NS)

    @functools.partial(
        pl.kernel,
        out_type=[
            jax.ShapeDtypeStruct((NPAD, D), jnp.float32),
            jax.ShapeDtypeStruct((NPAD, D), jnp.float32),
        ],
        mesh=mesh,
        scratch_types=[
            pltpu.VMEM_SHARED((NPAD, D), jnp.float32),
            pltpu.VMEM((16,), jnp.float32),
            pltpu.VMEM((PW,), jnp.float32),
            pltpu.VMEM((C, D), jnp.float32),
            pltpu.VMEM((C,), jnp.int32),
            pltpu.VMEM((1, 16), jnp.float32),
        ],
    )
    def k(wgt_h, m_h, seg_h, out0_h, out1_h, s_sh, mv, exs, wrows, segv, onev):
        c = lax.axis_index("c")
        s = lax.axis_index("s")
        zero16 = jnp.zeros((16,), jnp.float32)

        def z1(r, _):
            for j in range(D // 16):
                wrows[r, pl.ds(j * 16, 16)] = zero16
            return 0

        lax.fori_loop(0, C, z1, 0)
        onev[0, pl.ds(0, 16)] = jnp.ones((16,), jnp.float32)

        def z3(kk, _):
            r0 = s * (NPAD // NS) + kk * ZB
            pltpu.sync_copy(wrows.at[pl.ds(0, ZB)], s_sh.at[pl.ds(r0, ZB)])
            return 0

        lax.fori_loop(0, NPAD // NS // ZB, z3, 0)
        pltpu.sync_copy(m_h.at[pl.ds(0, 16)], mv)
        base = (s * NC + c) * PW
        pltpu.sync_copy(wgt_h.at[pl.ds(base, PW)], exs)
        mvv = mv[...]

        def expf(g, _):
            sl = pl.ds(g * 16, 16)
            exs[sl] = jnp.exp(exs[sl] - mvv)
            return 0

        lax.fori_loop(0, PW // 16, expf, 0)
        plsc.subcore_barrier()

        def pha(i, _):
            off = base + i * C
            pltpu.sync_copy(seg_h.at[pl.ds(off, C)], segv)

            def rowb(r, _):
                grpv = exs[pl.ds(i * C + (r // 16) * 16, 16)]
                bc = onev[0, pl.ds(0, 16)] * _bcast(grpv, r % 16)
                for j in range(D // 16):
                    wrows[r, pl.ds(j * 16, 16)] = bc
                return 0

            lax.fori_loop(0, C, rowb, 0)
            pltpu.sync_copy(wrows, s_sh.at[segv], add=True)
            return 0

        lax.fori_loop(0, nch, pha, 0)
        plsc.subcore_barrier()

        def dump(kk, _):
            r0 = s * (NPAD // NS) + kk * ZB
            pltpu.sync_copy(s_sh.at[pl.ds(r0, ZB)], wrows.at[pl.ds(0, ZB)])

            @pl.when(c == 0)
            def _():
                pltpu.sync_copy(wrows.at[pl.ds(0, ZB)], out0_h.at[pl.ds(r0, ZB)])

            @pl.when(c == 1)
            def _():
                pltpu.sync_copy(wrows.at[pl.ds(0, ZB)], out1_h.at[pl.ds(r0, ZB)])

            return 0

        lax.fori_loop(0, NPAD // NS // ZB, dump, 0)

    return k(wgt, mvec, seg)


@functools.partial(jax.jit, static_argnames=("C",))
def _sc_agg(wgt, mvec, X, seg, key, s0, s1, C):
    E = wgt.shape[0]
    PW = E // NW
    nch = PW // C
    mesh = plsc.VectorSubcoreMesh(core_axis_name="c", subcore_axis_name="s", num_cores=NC, num_subcores=NS)

    @functools.partial(
        pl.kernel,
        out_type=[
            jax.ShapeDtypeStruct((NPAD, D), jnp.float32),
            jax.ShapeDtypeStruct((NPAD, D), jnp.float32),
        ],
        mesh=mesh,
        scratch_types=[
            pltpu.VMEM_SHARED((NPAD, D), jnp.float32),
            pltpu.VMEM((16,), jnp.float32),
            pltpu.VMEM((PW,), jnp.float32),
            pltpu.VMEM((PW,), jnp.int32),
            pltpu.VMEM((C, D), jnp.float32),
            pltpu.VMEM((ZB, D), jnp.float32),
            pltpu.VMEM((ZB, D), jnp.float32),
            pltpu.VMEM((C,), jnp.int32),
            pltpu.SemaphoreType.DMA,
        ],
    )
    def k(wgt_h, m_h, X_h, seg_h, key_h, s0_h, s1_h, out0_h, out1_h,
          h_sh, mv, exs, keys, xrows, sa, sb, segv, sem):
        c = lax.axis_index("c")
        s = lax.axis_index("s")
        zero16 = jnp.zeros((16,), jnp.float32)

        def z1(r, _):
            for j in range(D // 16):
                xrows[r, pl.ds(j * 16, 16)] = zero16
            return 0

        lax.fori_loop(0, C, z1, 0)

        def z3(kk, _):
            r0 = s * (NPAD // NS) + kk * ZB
            pltpu.sync_copy(xrows.at[pl.ds(0, ZB)], h_sh.at[pl.ds(r0, ZB)])
            return 0

        lax.fori_loop(0, NPAD // NS // ZB, z3, 0)
        pltpu.sync_copy(m_h.at[pl.ds(0, 16)], mv)
        base = (s * NC + c) * PW
        pltpu.sync_copy(wgt_h.at[pl.ds(base, PW)], exs)
        pltpu.sync_copy(key_h.at[pl.ds(base, PW)], keys)
        mvv = mv[...]

        def expf(g, _):
            sl = pl.ds(g * 16, 16)
            exs[sl] = jnp.exp(exs[sl] - mvv)
            return 0

        lax.fori_loop(0, PW // 16, expf, 0)
        plsc.subcore_barrier()

        def phc(i, _):
            off = base + i * C
            pltpu.sync_copy(seg_h.at[pl.ds(off, C)], segv)
            pltpu.async_copy(
                X_h.at[keys.at[pl.ds(i * C, C)]], xrows, sem).wait()

            def rowf(r, _):
                grpv = exs[pl.ds(i * C + (r // 16) * 16, 16)]
                exb = _bcast(grpv, r % 16)
                for j in range(D // 16):
                    sl = (r, pl.ds(j * 16, 16))
                    xrows[sl] = xrows[sl] * exb
                return 0

            lax.fori_loop(0, C, rowf, 0)
            pltpu.sync_copy(xrows, h_sh.at[segv], add=True)
            return 0

        lax.fori_loop(0, nch, phc, 0)
        plsc.subcore_barrier()

        def dump(kk, _):
            r0 = s * (NPAD // NS) + kk * ZB
            pltpu.sync_copy(h_sh.at[pl.ds(r0, ZB)], xrows.at[pl.ds(0, ZB)])
            pltpu.sync_copy(s0_h.at[pl.ds(r0, ZB)], sa)
            pltpu.sync_copy(s1_h.at[pl.ds(r0, ZB)], sb)

            def nrm(r, _):
                for j in range(D // 16):
                    sl = (r, pl.ds(j * 16, 16))
                    xrows[sl] = xrows[sl] / (sa[sl] + sb[sl] + 1e-9)
                return 0

            lax.fori_loop(0, ZB, nrm, 0)

            @pl.when(c == 0)
            def _():
                pltpu.sync_copy(xrows.at[pl.ds(0, ZB)], out0_h.at[pl.ds(r0, ZB)])

            @pl.when(c == 1)
            def _():
                pltpu.sync_copy(xrows.at[pl.ds(0, ZB)], out1_h.at[pl.ds(r0, ZB)])

            return 0

        lax.fori_loop(0, NPAD // NS // ZB, dump, 0)

    return k(wgt, mvec, X, seg, key, s0, s1)


def _sc_softmax_agg(wgt, mvec, X, seg, key, C):
    s0, s1 = _sc_seg_sum(wgt, mvec, seg, C=C)
    return _sc_agg(wgt, mvec, X, seg, key, s0, s1, C=C)


def _jnp_softmax_agg(wgt, mvec, X, seg, key, C):
    ex = jnp.exp(wgt - mvec[0])
    s = jax.ops.segment_sum(ex, seg, num_segments=N_NODE)
    val = ex / (s[seg] + 1e-9)
    h = jax.ops.segment_sum(X[key] * val[:, None], seg, num_segments=N_NODE)
    pad = jnp.zeros((NPAD - N_NODE, D), jnp.float32)
    h = jnp.concatenate([h, pad], axis=0)
    return h, jnp.zeros_like(h)


def kernel(uv_src, uv_dst, uv_rating, uu_src, uu_dst, params):
    p = params
    relu = lambda x: jnp.maximum(x, 0.0)
    uv_src = uv_src.astype(jnp.int32)
    uv_dst = uv_dst.astype(jnp.int32)
    uv_rating = uv_rating.astype(jnp.int32)
    uu_src = uu_src.astype(jnp.int32)
    uu_dst = uu_dst.astype(jnp.int32)
    keyI = uv_rating * N_NODE + uv_dst
    keyU = uv_rating * N_NODE + uv_src

    x_ia_all, A_I = _pre_side(
        p['item_emb'], p['rating_emb'], p['gv_w1'], p['gv_b1'],
        p['gv_w2'], p['gv_b2'], p['attI_a1w'][:D], _lrelu)
    f_jt_all, A_U = _pre_side(
        p['user_emb'], p['rating_emb'], p['gu_w1'], p['gu_b1'],
        p['gu_w2'], p['gu_b2'], p['attU_a1w'][:D], relu)
    B_I = _mm([p['user_emb']], [p['attI_a1w'][D:]], p['attI_a1b'], None)
    B_U = _mm([p['item_emb']], [p['attU_a1w'][D:]], p['attU_a1b'], None)
    A_S = _mm([p['user_emb']], [p['attS_a1w'][:D]],
              jnp.zeros((D,), jnp.float32), None)

    t1 = _sc_gather_relu(A_I, B_I, keyI, uv_src, C=80)
    wgt, m = _att2(t1, p['attI_a2w'], p['attI_a2b'],
                   p['attI_a3w'], p['attI_a3b'])
    h0, h1 = _sc_softmax_agg(wgt, m, x_ia_all, uv_src, keyI, C=16)
    hI = _mm([h0, h1], [p['wi_w'], p['wi_w']], p['wi_b'], _lrelu)

    t1 = _sc_gather_relu(A_U, B_U, keyU, uv_dst, C=80)
    wgt, m = _att2(t1, p['attU_a2w'], p['attU_a2b'],
                   p['attU_a3w'], p['attU_a3b'])
    z0, z1 = _sc_softmax_agg(wgt, m, f_jt_all, uv_dst, keyU, C=16)
    z = _mm([z0, z1], [p['wu_w'], p['wu_w']], p['wu_b'], _lrelu)

    hI10k = hI[:N_NODE]
    B_S = _mm([hI10k], [p['attS_a1w'][D:]], p['attS_a1b'], None)
    t1 = _sc_gather_relu(A_S, B_S, uu_src, uu_dst, C=40)
    wgt, m = _att2(t1, p['attS_a2w'], p['attS_a2b'],
                   p['attS_a3w'], p['attS_a3b'])
    hs0, hs1 = _sc_softmax_agg(wgt, m, hI10k, uu_dst, uu_src, C=16)
    hS = _mm([hs0, hs1], [p['ws_w'], p['ws_w']], p['ws_b'], _lrelu)

    h_out = _mm([hI10k, hS[:N_NODE]], [p['w2_w'][:D], p['w2_w'][D:]],
                p['w2_b'], _lrelu)
    return (h_out, z[:N_NODE])


# speedup vs baseline: 4.1860x; 1.2418x over previous
import functools

import jax
import jax.numpy as jnp
from jax import lax
from jax.experimental import pallas as pl
from jax.experimental.pallas import tpu as pltpu
from jax.experimental.pallas import tpu_sc as plsc

N_NODE = 10000
D = 128
NR = 5
NV = NR * N_NODE
NC, NS = 2, 16
NW = NC * NS
NPAD = 10240
ZB = 16


def _lrelu(x):
    return jnp.where(x > 0, x, 0.2 * x)


def _bcast(vec16, j):
    idx = jnp.full((16, 1), j, jnp.int32)
    return lax.gather(
        vec16, idx,
        lax.GatherDimensionNumbers(
            offset_dims=(), collapsed_slice_dims=(0,), start_index_map=(0,)),
        (1,), mode=lax.GatherScatterMode.PROMISE_IN_BOUNDS)


def _pre_side(node_emb, r_emb, w1, b1, w2, b2, a1h, act):
    BM = 400
    nblk = N_NODE // BM

    def body(nb_ref, re_ref, w1_ref, b1_ref, w2_ref, b2_ref, a1_ref,
             x_ref, a_ref):
        w1a = w1_ref[:D, :]
        w1b = w1_ref[D:, :]
        rp = re_ref[...] @ w1b
        t = nb_ref[...] @ w1a
        b1v = b1_ref[...]
        for r in range(NR):
            x1 = act(t + rp[r][None, :] + b1v)
            xa = act(x1 @ w2_ref[...] + b2_ref[...])
            x_ref[r] = xa
            a_ref[r] = xa @ a1_ref[...]

    x_all, a_all = pl.pallas_call(
        body,
        grid=(nblk,),
        in_specs=[
            pl.BlockSpec((BM, D), lambda i: (i, 0)),
            pl.BlockSpec((NR, D), lambda i: (0, 0)),
            pl.BlockSpec((2 * D, D), lambda i: (0, 0)),
            pl.BlockSpec((1, D), lambda i: (0, 0)),
            pl.BlockSpec((D, D), lambda i: (0, 0)),
            pl.BlockSpec((1, D), lambda i: (0, 0)),
            pl.BlockSpec((D, D), lambda i: (0, 0)),
        ],
        out_specs=[
            pl.BlockSpec((NR, BM, D), lambda i: (0, i, 0)),
            pl.BlockSpec((NR, BM, D), lambda i: (0, i, 0)),
        ],
        out_shape=[
            jax.ShapeDtypeStruct((NR, N_NODE, D), jnp.float32),
            jax.ShapeDtypeStruct((NR, N_NODE, D), jnp.float32),
        ],
    )(node_emb, r_emb, w1, b1.reshape(1, D), w2, b2.reshape(1, D), a1h)
    return x_all.reshape(NV, D), a_all.reshape(NV, D)


def _mm(xs, ws, b, act):
    n = xs[0].shape[0]
    BM = 400 if n % 400 == 0 else 320
    nblk = n // BM
    nx = len(xs)

    def body(*refs):
        x_refs = refs[:nx]
        w_refs = refs[nx:2 * nx]
        b_ref = refs[2 * nx]
        o_ref = refs[2 * nx + 1]
        acc = b_ref[...]
        for xr, wr in zip(x_refs, w_refs):
            acc = acc + xr[...] @ wr[...]
        o_ref[...] = act(acc) if act is not None else acc

    in_specs = (
        [pl.BlockSpec((BM, x.shape[1]), lambda i: (i, 0)) for x in xs]
        + [pl.BlockSpec(w.shape, lambda i: (0, 0)) for w in ws]
        + [pl.BlockSpec((1, D), lambda i: (0, 0))]
    )
    return pl.pallas_call(
        body,
        grid=(nblk,),
        in_specs=in_specs,
        out_specs=pl.BlockSpec((BM, D), lambda i: (i, 0)),
        out_shape=jax.ShapeDtypeStruct((n, D), jnp.float32),
    )(*xs, *ws, b.reshape(1, D))


def _att2(t1, a2w, a2b, a3w, a3b):
    E = t1.shape[0]
    BM = 640
    nblk = E // BM

    def body(t1_ref, w_ref, b_ref, a3_ref, a3b_ref, w_out, m_out):
        i = pl.program_id(0)
        t2 = jnp.maximum(t1_ref[...] @ w_ref[...] + b_ref[...], 0.0)
        wv = jnp.sum(t2 * a3_ref[...], axis=1) + a3b_ref[0, 0]
        w_out[...] = wv.reshape(1, 8, BM // 8)

        @pl.when(i == 0)
        def _():
            m_out[...] = jnp.full((1, D), -1e30, jnp.float32)

        m_out[...] = jnp.maximum(m_out[...], jnp.max(wv))

    wgt, m = pl.pallas_call(
        body,
        grid=(nblk,),
        in_specs=[
            pl.BlockSpec((BM, D), lambda i: (i, 0)),
            pl.BlockSpec((D, D), lambda i: (0, 0)),
            pl.BlockSpec((1, D), lambda i: (0, 0)),
            pl.BlockSpec((1, D), lambda i: (0, 0)),
            pl.BlockSpec((1, 1), lambda i: (0, 0)),
        ],
        out_specs=[
            pl.BlockSpec((1, 8, BM // 8), lambda i: (i, 0, 0)),
            pl.BlockSpec((1, D), lambda i: (0, 0)),
        ],
        out_shape=[
            jax.ShapeDtypeStruct((nblk, 8, BM // 8), jnp.float32),
            jax.ShapeDtypeStruct((1, D), jnp.float32),
        ],
    )(t1, a2w, a2b.reshape(1, D), a3w.reshape(1, D), a3b.reshape(1, 1))
    return wgt.reshape(E), m.reshape(D)


@functools.partial(jax.jit, static_argnames=("C",))
def _sc_gather_relu(A, B, ia, ib, C):
    E = ia.shape[0]
    PW = E // NW
    nch = PW // C
    mesh = plsc.VectorSubcoreMesh(core_axis_name="c", subcore_axis_name="s", num_cores=NC, num_subcores=NS)

    @functools.partial(
        pl.kernel,
        out_type=jax.ShapeDtypeStruct((E, D), jnp.float32),
        mesh=mesh,
        scratch_types=[
            pltpu.VMEM((PW,), jnp.int32),
            pltpu.VMEM((PW,), jnp.int32),
            pltpu.VMEM((C, D), jnp.float32),
            pltpu.VMEM((C, D), jnp.float32),
            pltpu.SemaphoreType.DMA,
            pltpu.SemaphoreType.DMA,
        ],
    )
    def k(A_h, B_h, ia_h, ib_h, out_h, iav, ibv, rowsA, rowsB, semA, semB):
        c = lax.axis_index("c")
        s = lax.axis_index("s")
        base = (s * NC + c) * PW
        pltpu.sync_copy(ia_h.at[pl.ds(base, PW)], iav)
        pltpu.sync_copy(ib_h.at[pl.ds(base, PW)], ibv)

        def chunk(i, _):
            off = base + i * C
            cpA = pltpu.async_copy(A_h.at[iav.at[pl.ds(i * C, C)]], rowsA, semA)
            cpB = pltpu.async_copy(B_h.at[ibv.at[pl.ds(i * C, C)]], rowsB, semB)
            cpA.wait()
            cpB.wait()

            def rowf(r, _):
                for j in range(D // 16):
                    sl = (r, pl.ds(j * 16, 16))
                    rowsA[sl] = jnp.maximum(rowsA[sl] + rowsB[sl], 0.0)
                return 0

            lax.fori_loop(0, C, rowf, 0)
            pltpu.sync_copy(rowsA, out_h.at[pl.ds(off, C)])
            return 0

        lax.fori_loop(0, nch, chunk, 0)

    return k(A, B, ia, ib)


@functools.partial(jax.jit, static_argnames=("C",))
def _sc_seg_sum(wgt, mvec, seg, C):
    E = wgt.shape[0]
    PW = E // NW
    nch = PW // C
    mesh = plsc.VectorSubcoreMesh(core_axis_name="c", subcore_axis_name="s", num_cores=NC, num_subcores=NS)

    @functools.partial(
        pl.kernel,
        out_type=[
            jax.ShapeDtypeStruct((NPAD, D), jnp.float32),
            jax.ShapeDtypeStruct((NPAD, D), jnp.float32),
        ],
        mesh=mesh,
        scratch_types=[
            pltpu.VMEM_SHARED((NPAD, D), jnp.float32),
            pltpu.VMEM((16,), jnp.float32),
            pltpu.VMEM((PW,), jnp.float32),
            pltpu.VMEM((C, D), jnp.float32),
            pltpu.VMEM((C,), jnp.int32),
            pltpu.VMEM((1, 16), jnp.float32),
        ],
    )
    def k(wgt_h, m_h, seg_h, out0_h, out1_h, s_sh, mv, exs, wrows, segv, onev):
        c = lax.axis_index("c")
        s = lax.axis_index("s")
        zero16 = jnp.zeros((16,), jnp.float32)

        def z1(r, _):
            for j in range(D // 16):
                wrows[r, pl.ds(j * 16, 16)] = zero16
            return 0

        lax.fori_loop(0, C, z1, 0)
        onev[0, pl.ds(0, 16)] = jnp.ones((16,), jnp.float32)

        def z3(kk, _):
            r0 = s * (NPAD // NS) + kk * ZB
            pltpu.sync_copy(wrows.at[pl.ds(0, ZB)], s_sh.at[pl.ds(r0, ZB)])
            return 0

        lax.fori_loop(0, NPAD // NS // ZB, z3, 0)
        pltpu.sync_copy(m_h.at[pl.ds(0, 16)], mv)
        base = (s * NC + c) * PW
        pltpu.sync_copy(wgt_h.at[pl.ds(base, PW)], exs)
        mvv = mv[...]

        def expf(g, _):
            sl = pl.ds(g * 16, 16)
            exs[sl] = jnp.exp(exs[sl] - mvv)
            return 0

        lax.fori_loop(0, PW // 16, expf, 0)
        plsc.subcore_barrier()

        def pha(i, _):
            off = base + i * C
            pltpu.sync_copy(seg_h.at[pl.ds(off, C)], segv)

            def rowb(r, _):
                grpv = exs[pl.ds(i * C + (r // 16) * 16, 16)]
                bc = onev[0, pl.ds(0, 16)] * _bcast(grpv, r % 16)
                for j in range(D // 16):
                    wrows[r, pl.ds(j * 16, 16)] = bc
                return 0

            lax.fori_loop(0, C, rowb, 0)
            pltpu.sync_copy(wrows, s_sh.at[segv], add=True)
            return 0

        lax.fori_loop(0, nch, pha, 0)
        plsc.subcore_barrier()

        def dump(kk, _):
            r0 = s * (NPAD // NS) + kk * ZB
            pltpu.sync_copy(s_sh.at[pl.ds(r0, ZB)], wrows.at[pl.ds(0, ZB)])

            @pl.when(c == 0)
            def _():
                pltpu.sync_copy(wrows.at[pl.ds(0, ZB)], out0_h.at[pl.ds(r0, ZB)])

            @pl.when(c == 1)
            def _():
                pltpu.sync_copy(wrows.at[pl.ds(0, ZB)], out1_h.at[pl.ds(r0, ZB)])

            return 0

        lax.fori_loop(0, NPAD // NS // ZB, dump, 0)

    return k(wgt, mvec, seg)


@functools.partial(jax.jit, static_argnames=("C",))
def _sc_agg(wgt, mvec, X, seg, key, s0, s1, C):
    E = wgt.shape[0]
    PW = E // NW
    nch = PW // C
    mesh = plsc.VectorSubcoreMesh(core_axis_name="c", subcore_axis_name="s", num_cores=NC, num_subcores=NS)

    @functools.partial(
        pl.kernel,
        out_type=[
            jax.ShapeDtypeStruct((NPAD, D), jnp.float32),
            jax.ShapeDtypeStruct((NPAD, D), jnp.float32),
        ],
        mesh=mesh,
        scratch_types=[
            pltpu.VMEM_SHARED((NPAD, D), jnp.float32),
            pltpu.VMEM((16,), jnp.float32),
            pltpu.VMEM((PW,), jnp.float32),
            pltpu.VMEM((PW,), jnp.int32),
            pltpu.VMEM((C, D), jnp.float32),
            pltpu.VMEM((C, D), jnp.float32),
            pltpu.VMEM((ZB, D), jnp.float32),
            pltpu.VMEM((ZB, D), jnp.float32),
            pltpu.VMEM((C,), jnp.int32),
            pltpu.SemaphoreType.DMA,
            pltpu.SemaphoreType.DMA,
        ],
    )
    def k(wgt_h, m_h, X_h, seg_h, key_h, s0_h, s1_h, out0_h, out1_h,
          h_sh, mv, exs, keys, xrows, xrows2, sa, sb, segv, sem, sem2):
        c = lax.axis_index("c")
        s = lax.axis_index("s")
        zero16 = jnp.zeros((16,), jnp.float32)

        def z1(r, _):
            for j in range(D // 16):
                xrows[r, pl.ds(j * 16, 16)] = zero16
            return 0

        lax.fori_loop(0, C, z1, 0)

        def z3(kk, _):
            r0 = s * (NPAD // NS) + kk * ZB
            pltpu.sync_copy(xrows.at[pl.ds(0, ZB)], h_sh.at[pl.ds(r0, ZB)])
            return 0

        lax.fori_loop(0, NPAD // NS // ZB, z3, 0)
        pltpu.sync_copy(m_h.at[pl.ds(0, 16)], mv)
        base = (s * NC + c) * PW
        pltpu.sync_copy(wgt_h.at[pl.ds(base, PW)], exs)
        pltpu.sync_copy(key_h.at[pl.ds(base, PW)], keys)
        mvv = mv[...]

        def expf(g, _):
            sl = pl.ds(g * 16, 16)
            exs[sl] = jnp.exp(exs[sl] - mvv)
            return 0

        lax.fori_loop(0, PW // 16, expf, 0)
        plsc.subcore_barrier()

        def do_chunk(i, xr, sm):
            off = base + i * C
            pltpu.sync_copy(seg_h.at[pl.ds(off, C)], segv)
            pltpu.make_async_copy(
                X_h.at[keys.at[pl.ds(i * C, C)]], xr, sm).wait()

            def rowf(r, _):
                grpv = exs[pl.ds(i * C + (r // 16) * 16, 16)]
                exb = _bcast(grpv, r % 16)
                for j in range(D // 16):
                    sl = (r, pl.ds(j * 16, 16))
                    xr[sl] = xr[sl] * exb
                return 0

            lax.fori_loop(0, C, rowf, 0)
            pltpu.sync_copy(xr, h_sh.at[segv], add=True)

        def issue(i, xr, sm):
            pltpu.async_copy(X_h.at[keys.at[pl.ds(i * C, C)]], xr, sm)

        issue(0, xrows, sem)

        def phc2(kk, _):
            a = 2 * kk
            b = a + 1

            @pl.when(b < nch)
            def _():
                issue(b, xrows2, sem2)

            do_chunk(a, xrows, sem)

            @pl.when(a + 2 < nch)
            def _():
                issue(a + 2, xrows, sem)

            @pl.when(b < nch)
            def _():
                do_chunk(b, xrows2, sem2)

            return 0

        lax.fori_loop(0, (nch + 1) // 2, phc2, 0)
        plsc.subcore_barrier()

        def dump(kk, _):
            r0 = s * (NPAD // NS) + kk * ZB
            pltpu.sync_copy(h_sh.at[pl.ds(r0, ZB)], xrows.at[pl.ds(0, ZB)])
            pltpu.sync_copy(s0_h.at[pl.ds(r0, ZB)], sa)
            pltpu.sync_copy(s1_h.at[pl.ds(r0, ZB)], sb)

            def nrm(r, _):
                for j in range(D // 16):
                    sl = (r, pl.ds(j * 16, 16))
                    xrows[sl] = xrows[sl] / (sa[sl] + sb[sl] + 1e-9)
                return 0

            lax.fori_loop(0, ZB, nrm, 0)

            @pl.when(c == 0)
            def _():
                pltpu.sync_copy(xrows.at[pl.ds(0, ZB)], out0_h.at[pl.ds(r0, ZB)])

            @pl.when(c == 1)
            def _():
                pltpu.sync_copy(xrows.at[pl.ds(0, ZB)], out1_h.at[pl.ds(r0, ZB)])

            return 0

        lax.fori_loop(0, NPAD // NS // ZB, dump, 0)

    return k(wgt, mvec, X, seg, key, s0, s1)


def _sc_softmax_agg(wgt, mvec, X, seg, key, C):
    s0, s1 = _sc_seg_sum(wgt, mvec, seg, C=C)
    return _sc_agg(wgt, mvec, X, seg, key, s0, s1, C=C)


def _jnp_softmax_agg(wgt, mvec, X, seg, key, C):
    ex = jnp.exp(wgt - mvec[0])
    s = jax.ops.segment_sum(ex, seg, num_segments=N_NODE)
    val = ex / (s[seg] + 1e-9)
    h = jax.ops.segment_sum(X[key] * val[:, None], seg, num_segments=N_NODE)
    pad = jnp.zeros((NPAD - N_NODE, D), jnp.float32)
    h = jnp.concatenate([h, pad], axis=0)
    return h, jnp.zeros_like(h)


def kernel(uv_src, uv_dst, uv_rating, uu_src, uu_dst, params):
    p = params
    relu = lambda x: jnp.maximum(x, 0.0)
    uv_src = uv_src.astype(jnp.int32)
    uv_dst = uv_dst.astype(jnp.int32)
    uv_rating = uv_rating.astype(jnp.int32)
    uu_src = uu_src.astype(jnp.int32)
    uu_dst = uu_dst.astype(jnp.int32)
    keyI = uv_rating * N_NODE + uv_dst
    keyU = uv_rating * N_NODE + uv_src

    x_ia_all, A_I = _pre_side(
        p['item_emb'], p['rating_emb'], p['gv_w1'], p['gv_b1'],
        p['gv_w2'], p['gv_b2'], p['attI_a1w'][:D], _lrelu)
    f_jt_all, A_U = _pre_side(
        p['user_emb'], p['rating_emb'], p['gu_w1'], p['gu_b1'],
        p['gu_w2'], p['gu_b2'], p['attU_a1w'][:D], relu)
    B_I = _mm([p['user_emb']], [p['attI_a1w'][D:]], p['attI_a1b'], None)
    B_U = _mm([p['item_emb']], [p['attU_a1w'][D:]], p['attU_a1b'], None)
    A_S = _mm([p['user_emb']], [p['attS_a1w'][:D]],
              jnp.zeros((D,), jnp.float32), None)

    t1 = _sc_gather_relu(A_I, B_I, keyI, uv_src, C=80)
    wgt, m = _att2(t1, p['attI_a2w'], p['attI_a2b'],
                   p['attI_a3w'], p['attI_a3b'])
    h0, h1 = _sc_softmax_agg(wgt, m, x_ia_all, uv_src, keyI, C=16)
    hI = _mm([h0, h1], [p['wi_w'], p['wi_w']], p['wi_b'], _lrelu)

    t1 = _sc_gather_relu(A_U, B_U, keyU, uv_dst, C=80)
    wgt, m = _att2(t1, p['attU_a2w'], p['attU_a2b'],
                   p['attU_a3w'], p['attU_a3b'])
    z0, z1 = _sc_softmax_agg(wgt, m, f_jt_all, uv_dst, keyU, C=16)
    z = _mm([z0, z1], [p['wu_w'], p['wu_w']], p['wu_b'], _lrelu)

    hI10k = hI[:N_NODE]
    B_S = _mm([hI10k], [p['attS_a1w'][D:]], p['attS_a1b'], None)
    t1 = _sc_gather_relu(A_S, B_S, uu_src, uu_dst, C=40)
    wgt, m = _att2(t1, p['attS_a2w'], p['attS_a2b'],
                   p['attS_a3w'], p['attS_a3b'])
    npad_uu = 163840 - wgt.shape[0]
    wgt_p = jnp.concatenate([wgt, jnp.full((npad_uu,), -1e30, jnp.float32)])
    seg_p = jnp.concatenate([uu_dst, jnp.full((npad_uu,), N_NODE, jnp.int32)])
    key_p = jnp.concatenate([uu_src, jnp.zeros((npad_uu,), jnp.int32)])
    hs0, hs1 = _sc_softmax_agg(wgt_p, m, hI10k, seg_p, key_p, C=16)
    hS = _mm([hs0, hs1], [p['ws_w'], p['ws_w']], p['ws_b'], _lrelu)

    h_out = _mm([hI10k, hS[:N_NODE]], [p['w2_w'][:D], p['w2_w'][D:]],
                p['w2_b'], _lrelu)
    return (h_out, z[:N_NODE])


# speedup vs baseline: 4.4941x; 1.0736x over previous
import functools

import jax
import jax.numpy as jnp
from jax import lax
from jax.experimental import pallas as pl
from jax.experimental.pallas import tpu as pltpu
from jax.experimental.pallas import tpu_sc as plsc

N_NODE = 10000
D = 128
NR = 5
NV = NR * N_NODE
NC, NS = 2, 16
NW = NC * NS
NPAD = 10240
ZB = 16


def _lrelu(x):
    return jnp.where(x > 0, x, 0.2 * x)


def _bcast(vec16, j):
    idx = jnp.full((16, 1), j, jnp.int32)
    return lax.gather(
        vec16, idx,
        lax.GatherDimensionNumbers(
            offset_dims=(), collapsed_slice_dims=(0,), start_index_map=(0,)),
        (1,), mode=lax.GatherScatterMode.PROMISE_IN_BOUNDS)


def _pre_side(node_emb, r_emb, w1, b1, w2, b2, a1h, act):
    BM = 400
    nblk = N_NODE // BM

    def body(nb_ref, re_ref, w1_ref, b1_ref, w2_ref, b2_ref, a1_ref,
             x_ref, a_ref):
        w1a = w1_ref[:D, :]
        w1b = w1_ref[D:, :]
        rp = re_ref[...] @ w1b
        t = nb_ref[...] @ w1a
        b1v = b1_ref[...]
        for r in range(NR):
            x1 = act(t + rp[r][None, :] + b1v)
            xa = act(x1 @ w2_ref[...] + b2_ref[...])
            x_ref[r] = xa
            a_ref[r] = xa @ a1_ref[...]

    x_all, a_all = pl.pallas_call(
        body,
        grid=(nblk,),
        in_specs=[
            pl.BlockSpec((BM, D), lambda i: (i, 0)),
            pl.BlockSpec((NR, D), lambda i: (0, 0)),
            pl.BlockSpec((2 * D, D), lambda i: (0, 0)),
            pl.BlockSpec((1, D), lambda i: (0, 0)),
            pl.BlockSpec((D, D), lambda i: (0, 0)),
            pl.BlockSpec((1, D), lambda i: (0, 0)),
            pl.BlockSpec((D, D), lambda i: (0, 0)),
        ],
        out_specs=[
            pl.BlockSpec((NR, BM, D), lambda i: (0, i, 0)),
            pl.BlockSpec((NR, BM, D), lambda i: (0, i, 0)),
        ],
        out_shape=[
            jax.ShapeDtypeStruct((NR, N_NODE, D), jnp.float32),
            jax.ShapeDtypeStruct((NR, N_NODE, D), jnp.float32),
        ],
    )(node_emb, r_emb, w1, b1.reshape(1, D), w2, b2.reshape(1, D), a1h)
    return x_all.reshape(NV, D), a_all.reshape(NV, D)


def _mm(xs, ws, b, act):
    n = xs[0].shape[0]
    BM = 400 if n % 400 == 0 else 320
    nblk = n // BM
    nx = len(xs)

    def body(*refs):
        x_refs = refs[:nx]
        w_refs = refs[nx:2 * nx]
        b_ref = refs[2 * nx]
        o_ref = refs[2 * nx + 1]
        acc = b_ref[...]
        for xr, wr in zip(x_refs, w_refs):
            acc = acc + xr[...] @ wr[...]
        o_ref[...] = act(acc) if act is not None else acc

    in_specs = (
        [pl.BlockSpec((BM, x.shape[1]), lambda i: (i, 0)) for x in xs]
        + [pl.BlockSpec(w.shape, lambda i: (0, 0)) for w in ws]
        + [pl.BlockSpec((1, D), lambda i: (0, 0))]
    )
    return pl.pallas_call(
        body,
        grid=(nblk,),
        in_specs=in_specs,
        out_specs=pl.BlockSpec((BM, D), lambda i: (i, 0)),
        out_shape=jax.ShapeDtypeStruct((n, D), jnp.float32),
    )(*xs, *ws, b.reshape(1, D))


def _att2(t1, a2w, a2b, a3w, a3b):
    E = t1.shape[0]
    BM = 640
    nblk = E // BM

    def body(t1_ref, w_ref, b_ref, a3_ref, a3b_ref, w_out, m_out):
        i = pl.program_id(0)
        t2 = jnp.maximum(t1_ref[...] @ w_ref[...] + b_ref[...], 0.0)
        wv = jnp.sum(t2 * a3_ref[...], axis=1) + a3b_ref[0, 0]
        w_out[...] = wv.reshape(1, 8, BM // 8)

        @pl.when(i == 0)
        def _():
            m_out[...] = jnp.full((1, D), -1e30, jnp.float32)

        m_out[...] = jnp.maximum(m_out[...], jnp.max(wv))

    wgt, m = pl.pallas_call(
        body,
        grid=(nblk,),
        in_specs=[
            pl.BlockSpec((BM, D), lambda i: (i, 0)),
            pl.BlockSpec((D, D), lambda i: (0, 0)),
            pl.BlockSpec((1, D), lambda i: (0, 0)),
            pl.BlockSpec((1, D), lambda i: (0, 0)),
            pl.BlockSpec((1, 1), lambda i: (0, 0)),
        ],
        out_specs=[
            pl.BlockSpec((1, 8, BM // 8), lambda i: (i, 0, 0)),
            pl.BlockSpec((1, D), lambda i: (0, 0)),
        ],
        out_shape=[
            jax.ShapeDtypeStruct((nblk, 8, BM // 8), jnp.float32),
            jax.ShapeDtypeStruct((1, D), jnp.float32),
        ],
    )(t1, a2w, a2b.reshape(1, D), a3w.reshape(1, D), a3b.reshape(1, 1))
    return wgt.reshape(E), m.reshape(D)


@functools.partial(jax.jit, static_argnames=("C",))
def _sc_gather_relu(A, B, ia, ib, C):
    E = ia.shape[0]
    PW = E // NW
    nch = PW // C
    mesh = plsc.VectorSubcoreMesh(core_axis_name="c", subcore_axis_name="s", num_cores=NC, num_subcores=NS)

    @functools.partial(
        pl.kernel,
        out_type=jax.ShapeDtypeStruct((E, D), jnp.float32),
        mesh=mesh,
        scratch_types=[
            pltpu.VMEM((PW,), jnp.int32),
            pltpu.VMEM((PW,), jnp.int32),
            pltpu.VMEM((C, D), jnp.float32),
            pltpu.VMEM((C, D), jnp.float32),
            pltpu.VMEM((C, D), jnp.float32),
            pltpu.VMEM((C, D), jnp.float32),
            pltpu.SemaphoreType.DMA,
            pltpu.SemaphoreType.DMA,
            pltpu.SemaphoreType.DMA,
            pltpu.SemaphoreType.DMA,
        ],
    )
    def k(A_h, B_h, ia_h, ib_h, out_h, iav, ibv,
          rA0, rB0, rA1, rB1, sA0, sB0, sA1, sB1):
        c = lax.axis_index("c")
        s = lax.axis_index("s")
        base = (s * NC + c) * PW
        pltpu.sync_copy(ia_h.at[pl.ds(base, PW)], iav)
        pltpu.sync_copy(ib_h.at[pl.ds(base, PW)], ibv)

        def issue(i, rA, rB, sA, sB):
            pltpu.async_copy(A_h.at[iav.at[pl.ds(i * C, C)]], rA, sA)
            pltpu.async_copy(B_h.at[ibv.at[pl.ds(i * C, C)]], rB, sB)

        def do_chunk(i, rA, rB, sA, sB):
            pltpu.make_async_copy(
                A_h.at[iav.at[pl.ds(i * C, C)]], rA, sA).wait()
            pltpu.make_async_copy(
                B_h.at[ibv.at[pl.ds(i * C, C)]], rB, sB).wait()

            def rowf(r, _):
                for j in range(D // 16):
                    sl = (r, pl.ds(j * 16, 16))
                    rA[sl] = jnp.maximum(rA[sl] + rB[sl], 0.0)
                return 0

            lax.fori_loop(0, C, rowf, 0)
            pltpu.sync_copy(rA, out_h.at[pl.ds(base + i * C, C)])

        issue(0, rA0, rB0, sA0, sB0)

        def pipe(kk, _):
            a = 2 * kk
            b = a + 1

            @pl.when(b < nch)
            def _():
                issue(b, rA1, rB1, sA1, sB1)

            do_chunk(a, rA0, rB0, sA0, sB0)

            @pl.when(a + 2 < nch)
            def _():
                issue(a + 2, rA0, rB0, sA0, sB0)

            @pl.when(b < nch)
            def _():
                do_chunk(b, rA1, rB1, sA1, sB1)

            return 0

        lax.fori_loop(0, (nch + 1) // 2, pipe, 0)

    return k(A, B, ia, ib)


@functools.partial(jax.jit, static_argnames=("C",))
def _sc_seg_sum(wgt, mvec, seg, C):
    E = wgt.shape[0]
    PW = E // NW
    nch = PW // C
    mesh = plsc.VectorSubcoreMesh(core_axis_name="c", subcore_axis_name="s", num_cores=NC, num_subcores=NS)

    @functools.partial(
        pl.kernel,
        out_type=[
            jax.ShapeDtypeStruct((NPAD, D), jnp.float32),
            jax.ShapeDtypeStruct((NPAD, D), jnp.float32),
        ],
        mesh=mesh,
        scratch_types=[
            pltpu.VMEM_SHARED((NPAD, D), jnp.float32),
            pltpu.VMEM((16,), jnp.float32),
            pltpu.VMEM((PW,), jnp.float32),
            pltpu.VMEM((C, D), jnp.float32),
            pltpu.VMEM((C,), jnp.int32),
            pltpu.VMEM((1, 16), jnp.float32),
        ],
    )
    def k(wgt_h, m_h, seg_h, out0_h, out1_h, s_sh, mv, exs, wrows, segv, onev):
        c = lax.axis_index("c")
        s = lax.axis_index("s")
        zero16 = jnp.zeros((16,), jnp.float32)

        def z1(r, _):
            for j in range(D // 16):
                wrows[r, pl.ds(j * 16, 16)] = zero16
            return 0

        lax.fori_loop(0, C, z1, 0)
        onev[0, pl.ds(0, 16)] = jnp.ones((16,), jnp.float32)

        def z3(kk, _):
            r0 = s * (NPAD // NS) + kk * ZB
            pltpu.sync_copy(wrows.at[pl.ds(0, ZB)], s_sh.at[pl.ds(r0, ZB)])
            return 0

        lax.fori_loop(0, NPAD // NS // ZB, z3, 0)
        pltpu.sync_copy(m_h.at[pl.ds(0, 16)], mv)
        base = (s * NC + c) * PW
        pltpu.sync_copy(wgt_h.at[pl.ds(base, PW)], exs)
        mvv = mv[...]

        def expf(g, _):
            sl = pl.ds(g * 16, 16)
            exs[sl] = jnp.exp(exs[sl] - mvv)
            return 0

        lax.fori_loop(0, PW // 16, expf, 0)
        plsc.subcore_barrier()

        def pha(i, _):
            off = base + i * C
            pltpu.sync_copy(seg_h.at[pl.ds(off, C)], segv)

            def rowb(r, _):
                grpv = exs[pl.ds(i * C + (r // 16) * 16, 16)]
                bc = onev[0, pl.ds(0, 16)] * _bcast(grpv, r % 16)
                for j in range(D // 16):
                    wrows[r, pl.ds(j * 16, 16)] = bc
                return 0

            lax.fori_loop(0, C, rowb, 0)
            pltpu.sync_copy(wrows, s_sh.at[segv], add=True)
            return 0

        lax.fori_loop(0, nch, pha, 0)
        plsc.subcore_barrier()

        def dump(kk, _):
            r0 = s * (NPAD // NS) + kk * ZB
            pltpu.sync_copy(s_sh.at[pl.ds(r0, ZB)], wrows.at[pl.ds(0, ZB)])

            @pl.when(c == 0)
            def _():
                pltpu.sync_copy(wrows.at[pl.ds(0, ZB)], out0_h.at[pl.ds(r0, ZB)])

            @pl.when(c == 1)
            def _():
                pltpu.sync_copy(wrows.at[pl.ds(0, ZB)], out1_h.at[pl.ds(r0, ZB)])

            return 0

        lax.fori_loop(0, NPAD // NS // ZB, dump, 0)

    return k(wgt, mvec, seg)


@functools.partial(jax.jit, static_argnames=("C",))
def _sc_agg(wgt, mvec, X, seg, key, s0, s1, C):
    E = wgt.shape[0]
    PW = E // NW
    nch = PW // C
    mesh = plsc.VectorSubcoreMesh(core_axis_name="c", subcore_axis_name="s", num_cores=NC, num_subcores=NS)

    @functools.partial(
        pl.kernel,
        out_type=[
            jax.ShapeDtypeStruct((NPAD, D), jnp.float32),
            jax.ShapeDtypeStruct((NPAD, D), jnp.float32),
        ],
        mesh=mesh,
        scratch_types=[
            pltpu.VMEM_SHARED((NPAD, D), jnp.float32),
            pltpu.VMEM((16,), jnp.float32),
            pltpu.VMEM((PW,), jnp.float32),
            pltpu.VMEM((PW,), jnp.int32),
            pltpu.VMEM((C, D), jnp.float32),
            pltpu.VMEM((C, D), jnp.float32),
            pltpu.VMEM((ZB, D), jnp.float32),
            pltpu.VMEM((ZB, D), jnp.float32),
            pltpu.VMEM((C,), jnp.int32),
            pltpu.SemaphoreType.DMA,
            pltpu.SemaphoreType.DMA,
        ],
    )
    def k(wgt_h, m_h, X_h, seg_h, key_h, s0_h, s1_h, out0_h, out1_h,
          h_sh, mv, exs, keys, xrows, xrows2, sa, sb, segv, sem, sem2):
        c = lax.axis_index("c")
        s = lax.axis_index("s")
        zero16 = jnp.zeros((16,), jnp.float32)

        def z1(r, _):
            for j in range(D // 16):
                xrows[r, pl.ds(j * 16, 16)] = zero16
            return 0

        lax.fori_loop(0, C, z1, 0)

        def z3(kk, _):
            r0 = s * (NPAD // NS) + kk * ZB
            pltpu.sync_copy(xrows.at[pl.ds(0, ZB)], h_sh.at[pl.ds(r0, ZB)])
            return 0

        lax.fori_loop(0, NPAD // NS // ZB, z3, 0)
        pltpu.sync_copy(m_h.at[pl.ds(0, 16)], mv)
        base = (s * NC + c) * PW
        pltpu.sync_copy(wgt_h.at[pl.ds(base, PW)], exs)
        pltpu.sync_copy(key_h.at[pl.ds(base, PW)], keys)
        mvv = mv[...]

        def expf(g, _):
            sl = pl.ds(g * 16, 16)
            exs[sl] = jnp.exp(exs[sl] - mvv)
            return 0

        lax.fori_loop(0, PW // 16, expf, 0)
        plsc.subcore_barrier()

        def do_chunk(i, xr, sm):
            off = base + i * C
            pltpu.sync_copy(seg_h.at[pl.ds(off, C)], segv)
            pltpu.make_async_copy(
                X_h.at[keys.at[pl.ds(i * C, C)]], xr, sm).wait()

            def rowf(r, _):
                grpv = exs[pl.ds(i * C + (r // 16) * 16, 16)]
                exb = _bcast(grpv, r % 16)
                for j in range(D // 16):
                    sl = (r, pl.ds(j * 16, 16))
                    xr[sl] = xr[sl] * exb
                return 0

            lax.fori_loop(0, C, rowf, 0)
            pltpu.sync_copy(xr, h_sh.at[segv], add=True)

        def issue(i, xr, sm):
            pltpu.async_copy(X_h.at[keys.at[pl.ds(i * C, C)]], xr, sm)

        issue(0, xrows, sem)

        def phc2(kk, _):
            a = 2 * kk
            b = a + 1

            @pl.when(b < nch)
            def _():
                issue(b, xrows2, sem2)

            do_chunk(a, xrows, sem)

            @pl.when(a + 2 < nch)
            def _():
                issue(a + 2, xrows, sem)

            @pl.when(b < nch)
            def _():
                do_chunk(b, xrows2, sem2)

            return 0

        lax.fori_loop(0, (nch + 1) // 2, phc2, 0)
        plsc.subcore_barrier()

        def dump(kk, _):
            r0 = s * (NPAD // NS) + kk * ZB
            pltpu.sync_copy(h_sh.at[pl.ds(r0, ZB)], xrows.at[pl.ds(0, ZB)])
            pltpu.sync_copy(s0_h.at[pl.ds(r0, ZB)], sa)
            pltpu.sync_copy(s1_h.at[pl.ds(r0, ZB)], sb)

            def nrm(r, _):
                for j in range(D // 16):
                    sl = (r, pl.ds(j * 16, 16))
                    xrows[sl] = xrows[sl] / (sa[sl] + sb[sl] + 1e-9)
                return 0

            lax.fori_loop(0, ZB, nrm, 0)

            @pl.when(c == 0)
            def _():
                pltpu.sync_copy(xrows.at[pl.ds(0, ZB)], out0_h.at[pl.ds(r0, ZB)])

            @pl.when(c == 1)
            def _():
                pltpu.sync_copy(xrows.at[pl.ds(0, ZB)], out1_h.at[pl.ds(r0, ZB)])

            return 0

        lax.fori_loop(0, NPAD // NS // ZB, dump, 0)

    return k(wgt, mvec, X, seg, key, s0, s1)


def _sc_softmax_agg(wgt, mvec, X, seg, key, C):
    s0, s1 = _sc_seg_sum(wgt, mvec, seg, C=C)
    return _sc_agg(wgt, mvec, X, seg, key, s0, s1, C=C)


def _jnp_softmax_agg(wgt, mvec, X, seg, key, C):
    ex = jnp.exp(wgt - mvec[0])
    s = jax.ops.segment_sum(ex, seg, num_segments=N_NODE)
    val = ex / (s[seg] + 1e-9)
    h = jax.ops.segment_sum(X[key] * val[:, None], seg, num_segments=N_NODE)
    pad = jnp.zeros((NPAD - N_NODE, D), jnp.float32)
    h = jnp.concatenate([h, pad], axis=0)
    return h, jnp.zeros_like(h)


def kernel(uv_src, uv_dst, uv_rating, uu_src, uu_dst, params):
    p = params
    relu = lambda x: jnp.maximum(x, 0.0)
    uv_src = uv_src.astype(jnp.int32)
    uv_dst = uv_dst.astype(jnp.int32)
    uv_rating = uv_rating.astype(jnp.int32)
    uu_src = uu_src.astype(jnp.int32)
    uu_dst = uu_dst.astype(jnp.int32)
    keyI = uv_rating * N_NODE + uv_dst
    keyU = uv_rating * N_NODE + uv_src

    x_ia_all, A_I = _pre_side(
        p['item_emb'], p['rating_emb'], p['gv_w1'], p['gv_b1'],
        p['gv_w2'], p['gv_b2'], p['attI_a1w'][:D], _lrelu)
    f_jt_all, A_U = _pre_side(
        p['user_emb'], p['rating_emb'], p['gu_w1'], p['gu_b1'],
        p['gu_w2'], p['gu_b2'], p['attU_a1w'][:D], relu)
    B_I = _mm([p['user_emb']], [p['attI_a1w'][D:]], p['attI_a1b'], None)
    B_U = _mm([p['item_emb']], [p['attU_a1w'][D:]], p['attU_a1b'], None)
    A_S = _mm([p['user_emb']], [p['attS_a1w'][:D]],
              jnp.zeros((D,), jnp.float32), None)

    t1 = _sc_gather_relu(A_I, B_I, keyI, uv_src, C=80)
    wgt, m = _att2(t1, p['attI_a2w'], p['attI_a2b'],
                   p['attI_a3w'], p['attI_a3b'])
    h0, h1 = _sc_softmax_agg(wgt, m, x_ia_all, uv_src, keyI, C=16)
    hI = _mm([h0, h1], [p['wi_w'], p['wi_w']], p['wi_b'], _lrelu)

    t1 = _sc_gather_relu(A_U, B_U, keyU, uv_dst, C=80)
    wgt, m = _att2(t1, p['attU_a2w'], p['attU_a2b'],
                   p['attU_a3w'], p['attU_a3b'])
    z0, z1 = _sc_softmax_agg(wgt, m, f_jt_all, uv_dst, keyU, C=16)
    z = _mm([z0, z1], [p['wu_w'], p['wu_w']], p['wu_b'], _lrelu)

    hI10k = hI[:N_NODE]
    B_S = _mm([hI10k], [p['attS_a1w'][D:]], p['attS_a1b'], None)
    t1 = _sc_gather_relu(A_S, B_S, uu_src, uu_dst, C=40)
    wgt, m = _att2(t1, p['attS_a2w'], p['attS_a2b'],
                   p['attS_a3w'], p['attS_a3b'])
    npad_uu = 163840 - wgt.shape[0]
    wgt_p = jnp.concatenate([wgt, jnp.full((npad_uu,), -1e30, jnp.float32)])
    seg_p = jnp.concatenate([uu_dst, jnp.full((npad_uu,), N_NODE, jnp.int32)])
    key_p = jnp.concatenate([uu_src, jnp.zeros((npad_uu,), jnp.int32)])
    hs0, hs1 = _sc_softmax_agg(wgt_p, m, hI10k, seg_p, key_p, C=16)
    hS = _mm([hs0, hs1], [p['ws_w'], p['ws_w']], p['ws_b'], _lrelu)

    h_out = _mm([hI10k, hS[:N_NODE]], [p['w2_w'][:D], p['w2_w'][D:]],
                p['w2_b'], _lrelu)
    return (h_out, z[:N_NODE])


# speedup vs baseline: 4.4971x; 1.0007x over previous
import functools

import jax
import jax.numpy as jnp
from jax import lax
from jax.experimental import pallas as pl
from jax.experimental.pallas import tpu as pltpu
from jax.experimental.pallas import tpu_sc as plsc

N_NODE = 10000
D = 128
NR = 5
NV = NR * N_NODE
NC, NS = 2, 16
NW = NC * NS
NPAD = 10240
ZB = 16


def _lrelu(x):
    return jnp.where(x > 0, x, 0.2 * x)


def _bcast(vec16, j):
    idx = jnp.full((16, 1), j, jnp.int32)
    return lax.gather(
        vec16, idx,
        lax.GatherDimensionNumbers(
            offset_dims=(), collapsed_slice_dims=(0,), start_index_map=(0,)),
        (1,), mode=lax.GatherScatterMode.PROMISE_IN_BOUNDS)


def _pre_side(node_emb, r_emb, w1, b1, w2, b2, a1h, act):
    BM = 400
    nblk = N_NODE // BM

    def body(nb_ref, re_ref, w1_ref, b1_ref, w2_ref, b2_ref, a1_ref,
             x_ref, a_ref):
        w1a = w1_ref[:D, :]
        w1b = w1_ref[D:, :]
        rp = re_ref[...] @ w1b
        t = nb_ref[...] @ w1a
        b1v = b1_ref[...]
        for r in range(NR):
            x1 = act(t + rp[r][None, :] + b1v)
            xa = act(x1 @ w2_ref[...] + b2_ref[...])
            x_ref[r] = xa
            a_ref[r] = xa @ a1_ref[...]

    x_all, a_all = pl.pallas_call(
        body,
        grid=(nblk,),
        in_specs=[
            pl.BlockSpec((BM, D), lambda i: (i, 0)),
            pl.BlockSpec((NR, D), lambda i: (0, 0)),
            pl.BlockSpec((2 * D, D), lambda i: (0, 0)),
            pl.BlockSpec((1, D), lambda i: (0, 0)),
            pl.BlockSpec((D, D), lambda i: (0, 0)),
            pl.BlockSpec((1, D), lambda i: (0, 0)),
            pl.BlockSpec((D, D), lambda i: (0, 0)),
        ],
        out_specs=[
            pl.BlockSpec((NR, BM, D), lambda i: (0, i, 0)),
            pl.BlockSpec((NR, BM, D), lambda i: (0, i, 0)),
        ],
        out_shape=[
            jax.ShapeDtypeStruct((NR, N_NODE, D), jnp.float32),
            jax.ShapeDtypeStruct((NR, N_NODE, D), jnp.float32),
        ],
    )(node_emb, r_emb, w1, b1.reshape(1, D), w2, b2.reshape(1, D), a1h)
    return x_all.reshape(NV, D), a_all.reshape(NV, D)


def _mm(xs, ws, b, act):
    n = xs[0].shape[0]
    BM = 400 if n % 400 == 0 else 320
    nblk = n // BM
    nx = len(xs)

    def body(*refs):
        x_refs = refs[:nx]
        w_refs = refs[nx:2 * nx]
        b_ref = refs[2 * nx]
        o_ref = refs[2 * nx + 1]
        acc = b_ref[...]
        for xr, wr in zip(x_refs, w_refs):
            acc = acc + xr[...] @ wr[...]
        o_ref[...] = act(acc) if act is not None else acc

    in_specs = (
        [pl.BlockSpec((BM, x.shape[1]), lambda i: (i, 0)) for x in xs]
        + [pl.BlockSpec(w.shape, lambda i: (0, 0)) for w in ws]
        + [pl.BlockSpec((1, D), lambda i: (0, 0))]
    )
    return pl.pallas_call(
        body,
        grid=(nblk,),
        in_specs=in_specs,
        out_specs=pl.BlockSpec((BM, D), lambda i: (i, 0)),
        out_shape=jax.ShapeDtypeStruct((n, D), jnp.float32),
    )(*xs, *ws, b.reshape(1, D))


def _att2(t1, a2w, a2b, a3w, a3b):
    E = t1.shape[0]
    BM = 640
    nblk = E // BM

    def body(t1_ref, w_ref, b_ref, a3_ref, a3b_ref, w_out, m_out):
        i = pl.program_id(0)
        t2 = jnp.maximum(t1_ref[...] @ w_ref[...] + b_ref[...], 0.0)
        wv = jnp.sum(t2 * a3_ref[...], axis=1) + a3b_ref[0, 0]
        w_out[...] = wv.reshape(1, 8, BM // 8)

        @pl.when(i == 0)
        def _():
            m_out[...] = jnp.full((1, D), -1e30, jnp.float32)

        m_out[...] = jnp.maximum(m_out[...], jnp.max(wv))

    wgt, m = pl.pallas_call(
        body,
        grid=(nblk,),
        in_specs=[
            pl.BlockSpec((BM, D), lambda i: (i, 0)),
            pl.BlockSpec((D, D), lambda i: (0, 0)),
            pl.BlockSpec((1, D), lambda i: (0, 0)),
            pl.BlockSpec((1, D), lambda i: (0, 0)),
            pl.BlockSpec((1, 1), lambda i: (0, 0)),
        ],
        out_specs=[
            pl.BlockSpec((1, 8, BM // 8), lambda i: (i, 0, 0)),
            pl.BlockSpec((1, D), lambda i: (0, 0)),
        ],
        out_shape=[
            jax.ShapeDtypeStruct((nblk, 8, BM // 8), jnp.float32),
            jax.ShapeDtypeStruct((1, D), jnp.float32),
        ],
    )(t1, a2w, a2b.reshape(1, D), a3w.reshape(1, D), a3b.reshape(1, 1))
    return wgt.reshape(E), m.reshape(D)


@functools.partial(jax.jit, static_argnames=("C",))
def _sc_gather_relu(A, B, ia, ib, C):
    E = ia.shape[0]
    PW = E // NW
    nch = PW // C
    mesh = plsc.VectorSubcoreMesh(core_axis_name="c", subcore_axis_name="s", num_cores=NC, num_subcores=NS)

    @functools.partial(
        pl.kernel,
        out_type=jax.ShapeDtypeStruct((E, D), jnp.float32),
        mesh=mesh,
        scratch_types=[
            pltpu.VMEM((PW,), jnp.int32),
            pltpu.VMEM((PW,), jnp.int32),
            pltpu.VMEM((C, D), jnp.float32),
            pltpu.VMEM((C, D), jnp.float32),
            pltpu.VMEM((C, D), jnp.float32),
            pltpu.VMEM((C, D), jnp.float32),
            pltpu.SemaphoreType.DMA,
            pltpu.SemaphoreType.DMA,
            pltpu.SemaphoreType.DMA,
            pltpu.SemaphoreType.DMA,
        ],
    )
    def k(A_h, B_h, ia_h, ib_h, out_h, iav, ibv,
          rA0, rB0, rA1, rB1, sA0, sB0, sA1, sB1):
        c = lax.axis_index("c")
        s = lax.axis_index("s")
        base = (s * NC + c) * PW
        pltpu.sync_copy(ia_h.at[pl.ds(base, PW)], iav)
        pltpu.sync_copy(ib_h.at[pl.ds(base, PW)], ibv)

        def issue(i, rA, rB, sA, sB):
            pltpu.async_copy(A_h.at[iav.at[pl.ds(i * C, C)]], rA, sA)
            pltpu.async_copy(B_h.at[ibv.at[pl.ds(i * C, C)]], rB, sB)

        def do_chunk(i, rA, rB, sA, sB):
            pltpu.make_async_copy(
                A_h.at[iav.at[pl.ds(i * C, C)]], rA, sA).wait()
            pltpu.make_async_copy(
                B_h.at[ibv.at[pl.ds(i * C, C)]], rB, sB).wait()

            def rowf(r, _):
                for j in range(D // 16):
                    sl = (r, pl.ds(j * 16, 16))
                    rA[sl] = jnp.maximum(rA[sl] + rB[sl], 0.0)
                return 0

            lax.fori_loop(0, C, rowf, 0)
            pltpu.sync_copy(rA, out_h.at[pl.ds(base + i * C, C)])

        issue(0, rA0, rB0, sA0, sB0)

        def pipe(kk, _):
            a = 2 * kk
            b = a + 1

            @pl.when(b < nch)
            def _():
                issue(b, rA1, rB1, sA1, sB1)

            do_chunk(a, rA0, rB0, sA0, sB0)

            @pl.when(a + 2 < nch)
            def _():
                issue(a + 2, rA0, rB0, sA0, sB0)

            @pl.when(b < nch)
            def _():
                do_chunk(b, rA1, rB1, sA1, sB1)

            return 0

        lax.fori_loop(0, (nch + 1) // 2, pipe, 0)

    return k(A, B, ia, ib)


@functools.partial(jax.jit, static_argnames=("C",))
def _sc_seg_sum(wgt, mvec, seg, C):
    E = wgt.shape[0]
    PW = E // NW
    nch = PW // C
    mesh = plsc.VectorSubcoreMesh(core_axis_name="c", subcore_axis_name="s", num_cores=NC, num_subcores=NS)

    @functools.partial(
        pl.kernel,
        out_type=[
            jax.ShapeDtypeStruct((NPAD, D), jnp.float32),
            jax.ShapeDtypeStruct((NPAD, D), jnp.float32),
        ],
        mesh=mesh,
        scratch_types=[
            pltpu.VMEM_SHARED((NPAD, D), jnp.float32),
            pltpu.VMEM((16,), jnp.float32),
            pltpu.VMEM((PW,), jnp.float32),
            pltpu.VMEM((C, D), jnp.float32),
            pltpu.VMEM((C,), jnp.int32),
            pltpu.VMEM((1, 16), jnp.float32),
        ],
    )
    def k(wgt_h, m_h, seg_h, out0_h, out1_h, s_sh, mv, exs, wrows, segv, onev):
        c = lax.axis_index("c")
        s = lax.axis_index("s")
        zero16 = jnp.zeros((16,), jnp.float32)

        def z1(r, _):
            for j in range(D // 16):
                wrows[r, pl.ds(j * 16, 16)] = zero16
            return 0

        lax.fori_loop(0, C, z1, 0)
        onev[0, pl.ds(0, 16)] = jnp.ones((16,), jnp.float32)

        def z3(kk, _):
            r0 = s * (NPAD // NS) + kk * ZB
            pltpu.sync_copy(wrows.at[pl.ds(0, ZB)], s_sh.at[pl.ds(r0, ZB)])
            return 0

        lax.fori_loop(0, NPAD // NS // ZB, z3, 0)
        pltpu.sync_copy(m_h.at[pl.ds(0, 16)], mv)
        base = (s * NC + c) * PW
        pltpu.sync_copy(wgt_h.at[pl.ds(base, PW)], exs)
        mvv = mv[...]

        def expf(g, _):
            sl = pl.ds(g * 16, 16)
            exs[sl] = jnp.exp(exs[sl] - mvv)
            return 0

        lax.fori_loop(0, PW // 16, expf, 0)
        plsc.subcore_barrier()

        def pha(i, _):
            off = base + i * C
            pltpu.sync_copy(seg_h.at[pl.ds(off, C)], segv)

            def rowb(r, _):
                grpv = exs[pl.ds(i * C + (r // 16) * 16, 16)]
                bc = onev[0, pl.ds(0, 16)] * _bcast(grpv, r % 16)
                for j in range(D // 16):
                    wrows[r, pl.ds(j * 16, 16)] = bc
                return 0

            lax.fori_loop(0, C, rowb, 0)
            pltpu.sync_copy(wrows, s_sh.at[segv], add=True)
            return 0

        lax.fori_loop(0, nch, pha, 0)
        plsc.subcore_barrier()

        def dump(kk, _):
            r0 = s * (NPAD // NS) + kk * ZB
            pltpu.sync_copy(s_sh.at[pl.ds(r0, ZB)], wrows.at[pl.ds(0, ZB)])

            @pl.when(c == 0)
            def _():
                pltpu.sync_copy(wrows.at[pl.ds(0, ZB)], out0_h.at[pl.ds(r0, ZB)])

            @pl.when(c == 1)
            def _():
                pltpu.sync_copy(wrows.at[pl.ds(0, ZB)], out1_h.at[pl.ds(r0, ZB)])

            return 0

        lax.fori_loop(0, NPAD // NS // ZB, dump, 0)

    return k(wgt, mvec, seg)


@functools.partial(jax.jit, static_argnames=("C",))
def _sc_agg(wgt, mvec, X, seg, key, s0, s1, C):
    E = wgt.shape[0]
    PW = E // NW
    nch = PW // C
    mesh = plsc.VectorSubcoreMesh(core_axis_name="c", subcore_axis_name="s", num_cores=NC, num_subcores=NS)

    @functools.partial(
        pl.kernel,
        out_type=[
            jax.ShapeDtypeStruct((NPAD, D), jnp.float32),
            jax.ShapeDtypeStruct((NPAD, D), jnp.float32),
        ],
        mesh=mesh,
        scratch_types=[
            pltpu.VMEM_SHARED((NPAD, D), jnp.float32),
            pltpu.VMEM((16,), jnp.float32),
            pltpu.VMEM((PW,), jnp.float32),
            pltpu.VMEM((PW,), jnp.int32),
            pltpu.VMEM((C, D), jnp.float32),
            pltpu.VMEM((C, D), jnp.float32),
            pltpu.VMEM((ZB, D), jnp.float32),
            pltpu.VMEM((ZB, D), jnp.float32),
            pltpu.VMEM((C,), jnp.int32),
            pltpu.SemaphoreType.DMA,
            pltpu.SemaphoreType.DMA,
        ],
    )
    def k(wgt_h, m_h, X_h, seg_h, key_h, s0_h, s1_h, out0_h, out1_h,
          h_sh, mv, exs, keys, xrows, xrows2, sa, sb, segv, sem, sem2):
        c = lax.axis_index("c")
        s = lax.axis_index("s")
        zero16 = jnp.zeros((16,), jnp.float32)

        def z1(r, _):
            for j in range(D // 16):
                xrows[r, pl.ds(j * 16, 16)] = zero16
            return 0

        lax.fori_loop(0, C, z1, 0)

        def z3(kk, _):
            r0 = s * (NPAD // NS) + kk * ZB
            pltpu.sync_copy(xrows.at[pl.ds(0, ZB)], h_sh.at[pl.ds(r0, ZB)])
            return 0

        lax.fori_loop(0, NPAD // NS // ZB, z3, 0)
        pltpu.sync_copy(m_h.at[pl.ds(0, 16)], mv)
        base = (s * NC + c) * PW
        pltpu.sync_copy(wgt_h.at[pl.ds(base, PW)], exs)
        pltpu.sync_copy(key_h.at[pl.ds(base, PW)], keys)
        mvv = mv[...]

        def expf(g, _):
            sl = pl.ds(g * 16, 16)
            exs[sl] = jnp.exp(exs[sl] - mvv)
            return 0

        lax.fori_loop(0, PW // 16, expf, 0)
        plsc.subcore_barrier()

        def do_chunk(i, xr, sm):
            off = base + i * C
            pltpu.sync_copy(seg_h.at[pl.ds(off, C)], segv)
            pltpu.make_async_copy(
                X_h.at[keys.at[pl.ds(i * C, C)]], xr, sm).wait()

            def rowf(r, _):
                grpv = exs[pl.ds(i * C + (r // 16) * 16, 16)]
                exb = _bcast(grpv, r % 16)
                for j in range(D // 16):
                    sl = (r, pl.ds(j * 16, 16))
                    xr[sl] = xr[sl] * exb
                return 0

            lax.fori_loop(0, C, rowf, 0)
            pltpu.sync_copy(xr, h_sh.at[segv], add=True)

        def issue(i, xr, sm):
            pltpu.async_copy(X_h.at[keys.at[pl.ds(i * C, C)]], xr, sm)

        issue(0, xrows, sem)

        def phc2(kk, _):
            a = 2 * kk
            b = a + 1

            @pl.when(b < nch)
            def _():
                issue(b, xrows2, sem2)

            do_chunk(a, xrows, sem)

            @pl.when(a + 2 < nch)
            def _():
                issue(a + 2, xrows, sem)

            @pl.when(b < nch)
            def _():
                do_chunk(b, xrows2, sem2)

            return 0

        lax.fori_loop(0, (nch + 1) // 2, phc2, 0)
        plsc.subcore_barrier()

        def dump(kk, _):
            r0 = s * (NPAD // NS) + kk * ZB
            pltpu.sync_copy(h_sh.at[pl.ds(r0, ZB)], xrows.at[pl.ds(0, ZB)])
            pltpu.sync_copy(s0_h.at[pl.ds(r0, ZB)], sa)
            pltpu.sync_copy(s1_h.at[pl.ds(r0, ZB)], sb)

            def nrm(r, _):
                for j in range(D // 16):
                    sl = (r, pl.ds(j * 16, 16))
                    xrows[sl] = xrows[sl] / (sa[sl] + sb[sl] + 1e-9)
                return 0

            lax.fori_loop(0, ZB, nrm, 0)

            @pl.when(c == 0)
            def _():
                pltpu.sync_copy(xrows.at[pl.ds(0, ZB)], out0_h.at[pl.ds(r0, ZB)])

            @pl.when(c == 1)
            def _():
                pltpu.sync_copy(xrows.at[pl.ds(0, ZB)], out1_h.at[pl.ds(r0, ZB)])

            return 0

        lax.fori_loop(0, NPAD // NS // ZB, dump, 0)

    return k(wgt, mvec, X, seg, key, s0, s1)


def _sc_softmax_agg(wgt, mvec, X, seg, key, C):
    s0, s1 = _sc_seg_sum(wgt, mvec, seg, C=C)
    return _sc_agg(wgt, mvec, X, seg, key, s0, s1, C=C)


def kernel(uv_src, uv_dst, uv_rating, uu_src, uu_dst, params):
    p = params
    relu = lambda x: jnp.maximum(x, 0.0)
    uv_src = uv_src.astype(jnp.int32)
    uv_dst = uv_dst.astype(jnp.int32)
    uv_rating = uv_rating.astype(jnp.int32)
    uu_src = uu_src.astype(jnp.int32)
    uu_dst = uu_dst.astype(jnp.int32)
    keyI = uv_rating * N_NODE + uv_dst
    keyU = uv_rating * N_NODE + uv_src

    x_ia_all, A_I = _pre_side(
        p['item_emb'], p['rating_emb'], p['gv_w1'], p['gv_b1'],
        p['gv_w2'], p['gv_b2'], p['attI_a1w'][:D], _lrelu)
    f_jt_all, A_U = _pre_side(
        p['user_emb'], p['rating_emb'], p['gu_w1'], p['gu_b1'],
        p['gu_w2'], p['gu_b2'], p['attU_a1w'][:D], relu)
    B_I = _mm([p['user_emb']], [p['attI_a1w'][D:]], p['attI_a1b'], None)
    B_U = _mm([p['item_emb']], [p['attU_a1w'][D:]], p['attU_a1b'], None)
    A_S = _mm([p['user_emb']], [p['attS_a1w'][:D]],
              jnp.zeros((D,), jnp.float32), None)

    t1 = _sc_gather_relu(A_I, B_I, keyI, uv_src, C=80)
    wgt, m = _att2(t1, p['attI_a2w'], p['attI_a2b'],
                   p['attI_a3w'], p['attI_a3b'])
    h0, h1 = _sc_softmax_agg(wgt, m, x_ia_all, uv_src, keyI, C=16)
    hI = _mm([h0, h1], [p['wi_w'], p['wi_w']], p['wi_b'], _lrelu)

    t1 = _sc_gather_relu(A_U, B_U, keyU, uv_dst, C=80)
    wgt, m = _att2(t1, p['attU_a2w'], p['attU_a2b'],
                   p['attU_a3w'], p['attU_a3b'])
    z0, z1 = _sc_softmax_agg(wgt, m, f_jt_all, uv_dst, keyU, C=16)
    z = _mm([z0, z1], [p['wu_w'], p['wu_w']], p['wu_b'], _lrelu)

    hI10k = hI[:N_NODE]
    B_S = _mm([hI10k], [p['attS_a1w'][D:]], p['attS_a1b'], None)
    t1 = _sc_gather_relu(A_S, B_S, uu_src, uu_dst, C=40)
    wgt, m = _att2(t1, p['attS_a2w'], p['attS_a2b'],
                   p['attS_a3w'], p['attS_a3b'])
    npad_uu = 163840 - wgt.shape[0]
    wgt_p = jnp.concatenate([wgt, jnp.full((npad_uu,), -1e30, jnp.float32)])
    seg_p = jnp.concatenate([uu_dst, jnp.full((npad_uu,), N_NODE, jnp.int32)])
    key_p = jnp.concatenate([uu_src, jnp.zeros((npad_uu,), jnp.int32)])
    hs0, hs1 = _sc_softmax_agg(wgt_p, m, hI10k, seg_p, key_p, C=16)
    hS = _mm([hs0, hs1], [p['ws_w'], p['ws_w']], p['ws_b'], _lrelu)

    h_out = _mm([hI10k, hS[:N_NODE]], [p['w2_w'][:D], p['w2_w'][D:]],
                p['w2_b'], _lrelu)
    return (h_out, z[:N_NODE])
